# Initial kernel scaffold; baseline (speedup 1.0000x reference)
import jax
import jax.numpy as jnp
from jax.experimental import pallas as pl


def kernel(x, edge_index, edge_attr, params):
    raise NotImplementedError("write your pallas kernel here")


# speedup vs baseline: 4.9379x; 4.9379x over previous
import functools

import jax
import jax.numpy as jnp
import numpy as np
from jax import lax
from jax.experimental import pallas as pl
from jax.experimental.pallas import tpu as pltpu
from jax.experimental.pallas import tpu_sc as plsc

N = 10000
E = 320000
D = 128
DE = 16
EH = 16

NC = 2
NS = 16
NW = NC * NS
EPW = E // NW
CH = 80
NCHUNK = EPW // CH
NPAD = 10240
SL = NPAD // NS
NROW = N // NS
SCALE = 1.0 / float(np.sqrt(D))
NEG = float("-inf")

_mesh = plsc.VectorSubcoreMesh(
    core_axis_name="c", subcore_axis_name="s", num_cores=NC, num_subcores=NS)

f32 = jnp.float32
i32 = jnp.int32


def _edge_embed(edge_attr, W, b):
    BE = 2000

    def body(ea_ref, w_ref, b_ref, o_ref):
        o_ref[...] = jnp.dot(ea_ref[...], w_ref[...],
                             preferred_element_type=f32) + b_ref[...]

    return pl.pallas_call(
        body,
        grid=(E // BE,),
        in_specs=[pl.BlockSpec((BE, DE), lambda i: (i, 0)),
                  pl.BlockSpec((DE, EH), lambda i: (0, 0)),
                  pl.BlockSpec((1, EH), lambda i: (0, 0))],
        out_specs=pl.BlockSpec((BE, EH), lambda i: (i, 0)),
        out_shape=jax.ShapeDtypeStruct((E, EH), f32),
    )(edge_attr, W, b.reshape(1, EH))


def _proj(h, p):
    BN = 400

    def body(h_ref, wq, bq, wk, bk, wv, bv, ws, bs, we,
             q_ref, qe_ref, k_ref, v_ref, s_ref):
        hh = h_ref[...]
        q = jnp.dot(hh, wq[...], preferred_element_type=f32) + bq[...]
        q_ref[...] = q
        qe_ref[...] = lax.dot_general(q, we[...], (((1,), (1,)), ((), ())),
                                      preferred_element_type=f32)
        k_ref[...] = jnp.dot(hh, wk[...], preferred_element_type=f32) + bk[...]
        v_ref[...] = jnp.dot(hh, wv[...], preferred_element_type=f32) + bv[...]
        s_ref[...] = jnp.dot(hh, ws[...], preferred_element_type=f32) + bs[...]

    row = lambda i: (i, 0)
    fix = lambda i: (0, 0)
    return pl.pallas_call(
        body,
        grid=(N // BN,),
        in_specs=[pl.BlockSpec((BN, D), row),
                  pl.BlockSpec((D, D), fix), pl.BlockSpec((1, D), fix),
                  pl.BlockSpec((D, D), fix), pl.BlockSpec((1, D), fix),
                  pl.BlockSpec((D, D), fix), pl.BlockSpec((1, D), fix),
                  pl.BlockSpec((D, D), fix), pl.BlockSpec((1, D), fix),
                  pl.BlockSpec((EH, D), fix)],
        out_specs=[pl.BlockSpec((BN, D), row), pl.BlockSpec((BN, EH), row),
                   pl.BlockSpec((BN, D), row), pl.BlockSpec((BN, D), row),
                   pl.BlockSpec((BN, D), row)],
        out_shape=[jax.ShapeDtypeStruct((N, D), f32),
                   jax.ShapeDtypeStruct((N, EH), f32),
                   jax.ShapeDtypeStruct((N, D), f32),
                   jax.ShapeDtypeStruct((N, D), f32),
                   jax.ShapeDtypeStruct((N, D), f32)],
    )(h, p['Wq'], p['bq'].reshape(1, D), p['Wk'], p['bk'].reshape(1, D),
      p['Wv'], p['bv'].reshape(1, D), p['Ws'], p['bs'].reshape(1, D), p['We'])


def _combine(vp, ag, skip, we, relu):
    BN = 400

    def body(vp_ref, ag_ref, s_ref, we_ref, o_ref):
        h = vp_ref[0] + vp_ref[1] + s_ref[...]
        agg = ag_ref[0] + ag_ref[1]
        h = h + jnp.dot(agg, we_ref[...], preferred_element_type=f32)
        if relu:
            h = jnp.maximum(h, 0.0)
        o_ref[...] = h

    return pl.pallas_call(
        body,
        grid=(N // BN,),
        in_specs=[pl.BlockSpec((2, BN, D), lambda i: (0, i, 0)),
                  pl.BlockSpec((2, BN, EH), lambda i: (0, i, 0)),
                  pl.BlockSpec((BN, D), lambda i: (i, 0)),
                  pl.BlockSpec((EH, D), lambda i: (0, 0))],
        out_specs=pl.BlockSpec((BN, D), lambda i: (i, 0)),
        out_shape=jax.ShapeDtypeStruct((N, D), f32),
    )(vp, ag, skip, we)


@functools.partial(
    pl.kernel,
    out_type=[jax.ShapeDtypeStruct((E,), f32),
              jax.ShapeDtypeStruct((NC, NPAD), f32)],
    mesh=_mesh,
    compiler_params=pltpu.CompilerParams(needs_layout_passes=False, use_tc_tiling_on_sc=False),
    scratch_types=[
        pltpu.VMEM((CH,), i32),
        pltpu.VMEM((CH,), i32),
        pltpu.VMEM((CH, D), f32),
        pltpu.VMEM((CH, D), f32),
        pltpu.VMEM((CH, EH), f32),
        pltpu.VMEM((CH, EH), f32),
        pltpu.VMEM((CH,), f32),
        pltpu.VMEM((NPAD,), f32),
        pltpu.VMEM((SL,), f32),
        pltpu.VMEM((SL,), f32),
        pltpu.VMEM_SHARED((NS, NPAD), f32),
    ],
)
def _sc_alpha(q_hbm, qe_hbm, k_hbm, e_hbm, src_hbm, dst_hbm,
              alpha_hbm, mpart_hbm,
              idx_s, idx_d, qrows, krows, qerows, erows, abuf, mloc,
              acc, tmp, msh):
    c = lax.axis_index("c")
    s = lax.axis_index("s")
    wid = c * NS + s
    base = wid * EPW

    def mi(i, carry):
        mloc[pl.ds(i * 16, 16)] = jnp.full((16,), NEG, f32)
        return carry
    lax.fori_loop(0, NPAD // 16, mi, 0)

    def chunk(ci, carry):
        off = base + ci * CH
        pltpu.sync_copy(src_hbm.at[pl.ds(off, CH)], idx_s)
        pltpu.sync_copy(dst_hbm.at[pl.ds(off, CH)], idx_d)
        pltpu.sync_copy(e_hbm.at[pl.ds(off, CH)], erows)
        pltpu.sync_copy(k_hbm.at[idx_s], krows)
        pltpu.sync_copy(q_hbm.at[idx_d], qrows)
        pltpu.sync_copy(qe_hbm.at[idx_d], qerows)

        lane = lax.broadcasted_iota(i32, (16,), 0)

        def grp(g, gcarry):
            def edge(j, accv):
                jj = g * 16 + j
                av = qrows[jj, pl.ds(0, 16)] * krows[jj, pl.ds(0, 16)]
                for t in range(1, D // 16):
                    av = av + (qrows[jj, pl.ds(16 * t, 16)]
                               * krows[jj, pl.ds(16 * t, 16)])
                av = av + qerows[jj, :] * erows[jj, :]
                for sh in (8, 4, 2, 1):
                    perm = jnp.bitwise_xor(lane, sh)
                    av = av + av.at[perm].get(mode='promise_in_bounds')
                return jnp.where(lane == j, av * SCALE, accv)
            a16 = lax.fori_loop(0, 16, edge, jnp.zeros((16,), f32))
            abuf[pl.ds(g * 16, 16)] = a16
            d16 = idx_d[pl.ds(g * 16, 16)]

            def bd(t, pend):
                msk = pend > 0
                old = plsc.load_gather(mloc, [d16])
                new = jnp.maximum(old, a16)
                plsc.store_scatter(mloc, [d16], new, mask=msk)
                chk = plsc.load_gather(mloc, [d16])
                return (msk & (chk < new)).astype(i32)
            lax.fori_loop(0, 16, bd, jnp.ones((16,), i32))
            return gcarry
        lax.fori_loop(0, CH // 16, grp, 0)
        pltpu.sync_copy(abuf, alpha_hbm.at[pl.ds(off, CH)])
        return carry
    lax.fori_loop(0, NCHUNK, chunk, 0)

    pltpu.sync_copy(mloc, msh.at[s])
    plsc.subcore_barrier()
    col = s * SL
    pltpu.sync_copy(msh.at[0, pl.ds(col, SL)], acc)
    for t in range(1, NS):
        pltpu.sync_copy(msh.at[t, pl.ds(col, SL)], tmp)

        def mx(i, carry):
            acc[pl.ds(i * 16, 16)] = jnp.maximum(acc[pl.ds(i * 16, 16)],
                                                 tmp[pl.ds(i * 16, 16)])
            return carry
        lax.fori_loop(0, SL // 16, mx, 0)
    pltpu.sync_copy(acc, mpart_hbm.at[c, pl.ds(col, SL)])


@functools.partial(
    pl.kernel,
    out_type=[jax.ShapeDtypeStruct((E,), f32),
              jax.ShapeDtypeStruct((NC, NPAD), f32)],
    mesh=_mesh,
    compiler_params=pltpu.CompilerParams(needs_layout_passes=False, use_tc_tiling_on_sc=False),
    scratch_types=[
        pltpu.VMEM((NPAD,), f32),
        pltpu.VMEM((NPAD,), f32),
        pltpu.VMEM((CH,), i32),
        pltpu.VMEM((CH,), f32),
        pltpu.VMEM((CH,), f32),
        pltpu.VMEM((SL,), f32),
        pltpu.VMEM_SHARED((NPAD,), f32),
    ],
)
def _sc_exden(alpha_hbm, dst_hbm, mpart_hbm,
              ex_hbm, dpart_hbm,
              b0, b1, idx_d, abuf, exbuf, zb, dsh):
    c = lax.axis_index("c")
    s = lax.axis_index("s")
    wid = c * NS + s
    base = wid * EPW

    pltpu.sync_copy(mpart_hbm.at[0], b0)
    pltpu.sync_copy(mpart_hbm.at[1], b1)

    def mcomb(i, carry):
        m = jnp.maximum(b0[pl.ds(i * 16, 16)], b1[pl.ds(i * 16, 16)])
        fin = (m - m) == 0.0
        b0[pl.ds(i * 16, 16)] = jnp.where(fin, m, 0.0)
        return carry
    lax.fori_loop(0, NPAD // 16, mcomb, 0)

    def z(i, carry):
        zb[pl.ds(i * 16, 16)] = jnp.zeros((16,), f32)
        return carry
    lax.fori_loop(0, SL // 16, z, 0)
    col = s * SL
    pltpu.sync_copy(zb, dsh.at[pl.ds(col, SL)])
    plsc.subcore_barrier()

    def chunk(ci, carry):
        off = base + ci * CH
        pltpu.sync_copy(dst_hbm.at[pl.ds(off, CH)], idx_d)
        pltpu.sync_copy(alpha_hbm.at[pl.ds(off, CH)], abuf)

        def grp(g, gcarry):
            d16 = idx_d[pl.ds(g * 16, 16)]
            m16 = plsc.load_gather(b0, [d16])
            exbuf[pl.ds(g * 16, 16)] = jnp.exp(abuf[pl.ds(g * 16, 16)] - m16)
            return gcarry
        lax.fori_loop(0, CH // 16, grp, 0)
        pltpu.sync_copy(exbuf, ex_hbm.at[pl.ds(off, CH)])
        pltpu.sync_copy(exbuf, dsh.at[idx_d], add=True)
        return carry
    lax.fori_loop(0, NCHUNK, chunk, 0)

    plsc.subcore_barrier()
    pltpu.sync_copy(dsh.at[pl.ds(col, SL)], dpart_hbm.at[c, pl.ds(col, SL)])


@functools.partial(
    pl.kernel,
    out_type=[jax.ShapeDtypeStruct((NC, N, D), f32),
              jax.ShapeDtypeStruct((NC, N, EH), f32)],
    mesh=_mesh,
    compiler_params=pltpu.CompilerParams(needs_layout_passes=False, use_tc_tiling_on_sc=False),
    scratch_types=[
        pltpu.VMEM((NPAD,), f32),
        pltpu.VMEM((NPAD,), f32),
        pltpu.VMEM((CH,), i32),
        pltpu.VMEM((CH,), i32),
        pltpu.VMEM((CH, D), f32),
        pltpu.VMEM((CH, EH), f32),
        pltpu.VMEM((CH,), f32),
        pltpu.VMEM((CH,), f32),
        pltpu.VMEM_SHARED((N, D), f32),
        pltpu.VMEM_SHARED((N, EH), f32),
    ],
)
def _sc_agg(ex_hbm, dpart_hbm, v_hbm, e_hbm, src_hbm, dst_hbm,
            vpart_hbm, agpart_hbm,
            d0, d1, idx_s, idx_d, vrows, erows, exbuf, abuf,
            vacc, agacc):
    c = lax.axis_index("c")
    s = lax.axis_index("s")
    wid = c * NS + s
    base = wid * EPW

    pltpu.sync_copy(dpart_hbm.at[0], d0)
    pltpu.sync_copy(dpart_hbm.at[1], d1)

    def dcomb(i, carry):
        d0[pl.ds(i * 16, 16)] = (d0[pl.ds(i * 16, 16)]
                                 + d1[pl.ds(i * 16, 16)] + 1e-16)
        return carry
    lax.fori_loop(0, NPAD // 16, dcomb, 0)

    def zrow(i, carry):
        for t in range(D // 16):
            vrows[i, pl.ds(16 * t, 16)] = jnp.zeros((16,), f32)
        erows[i, :] = jnp.zeros((16,), f32)
        return carry
    lax.fori_loop(0, CH, zrow, 0)

    row0 = s * NROW
    for (st, cnt) in ((0, 80), (80, 80), (160, 80), (240, 80),
                      (320, 80), (400, 80), (480, 80), (560, 65)):
        pltpu.sync_copy(vrows.at[pl.ds(0, cnt)], vacc.at[pl.ds(row0 + st, cnt)])
        pltpu.sync_copy(erows.at[pl.ds(0, cnt)], agacc.at[pl.ds(row0 + st, cnt)])
    plsc.subcore_barrier()

    def chunk(ci, carry):
        off = base + ci * CH
        pltpu.sync_copy(src_hbm.at[pl.ds(off, CH)], idx_s)
        pltpu.sync_copy(dst_hbm.at[pl.ds(off, CH)], idx_d)
        pltpu.sync_copy(e_hbm.at[pl.ds(off, CH)], erows)
        pltpu.sync_copy(ex_hbm.at[pl.ds(off, CH)], exbuf)
        pltpu.sync_copy(v_hbm.at[idx_s], vrows)

        def grp(g, gcarry):
            d16 = idx_d[pl.ds(g * 16, 16)]
            den = plsc.load_gather(d0, [d16])
            a16 = exbuf[pl.ds(g * 16, 16)] / den

            def edge(l, ecarry):
                jj = g * 16 + l
                idx = lax.broadcast(l, (16,))
                aj = a16.at[idx].get(mode='promise_in_bounds')
                for t in range(D // 16):
                    vrows[jj, pl.ds(16 * t, 16)] = (
                        vrows[jj, pl.ds(16 * t, 16)] * aj)
                erows[jj, :] = erows[jj, :] * aj
                return ecarry
            lax.fori_loop(0, 16, edge, 0)
            return gcarry
        lax.fori_loop(0, CH // 16, grp, 0)

        pltpu.sync_copy(vrows, vacc.at[idx_d], add=True)
        pltpu.sync_copy(erows, agacc.at[idx_d], add=True)
        return carry
    lax.fori_loop(0, NCHUNK, chunk, 0)

    plsc.subcore_barrier()
    pltpu.sync_copy(vacc.at[pl.ds(row0, NROW)],
                    vpart_hbm.at[c, pl.ds(row0, NROW)])
    pltpu.sync_copy(agacc.at[pl.ds(row0, NROW)],
                    agpart_hbm.at[c, pl.ds(row0, NROW)])


def kernel(x, edge_index, edge_attr, params):
    src = edge_index[0].astype(i32)
    dst = edge_index[1].astype(i32)
    e = _edge_embed(edge_attr, params['W_emb'], params['b_emb'])
    h = x
    n_layers = len(params['layers'])
    for li, p in enumerate(params['layers']):
        q, qe, k, v, skip = _proj(h, p)
        alpha, mpart = _sc_alpha(q, qe, k, e, src, dst)
        ex, dpart = _sc_exden(alpha, dst, mpart)
        vpart, agpart = _sc_agg(ex, dpart, v, e, src, dst)
        h = _combine(vpart, agpart, skip, p['We'], li < n_layers - 1)
    return h


# speedup vs baseline: 6.3816x; 1.2924x over previous
import functools

import jax
import jax.numpy as jnp
import numpy as np
from jax import lax
from jax.experimental import pallas as pl
from jax.experimental.pallas import tpu as pltpu
from jax.experimental.pallas import tpu_sc as plsc

N = 10000
E = 320000
D = 128
DE = 16
EH = 16

NC = 2
NS = 16
NW = NC * NS
EPW = E // NW
CH = 80
NCHUNK = EPW // CH
NPAD = 10240
SL = NPAD // NS
NROW = N // NS
SCALE = 1.0 / float(np.sqrt(D))
NEG = float("-inf")

_mesh = plsc.VectorSubcoreMesh(
    core_axis_name="c", subcore_axis_name="s", num_cores=NC, num_subcores=NS)

f32 = jnp.float32
i32 = jnp.int32


def _edge_embed(edge_attr, W, b):
    BE = 2000

    def body(ea_ref, w_ref, b_ref, o_ref):
        o_ref[...] = jnp.dot(ea_ref[...], w_ref[...],
                             preferred_element_type=f32) + b_ref[...]

    return pl.pallas_call(
        body,
        grid=(E // BE,),
        in_specs=[pl.BlockSpec((BE, DE), lambda i: (i, 0)),
                  pl.BlockSpec((DE, EH), lambda i: (0, 0)),
                  pl.BlockSpec((1, EH), lambda i: (0, 0))],
        out_specs=pl.BlockSpec((BE, EH), lambda i: (i, 0)),
        out_shape=jax.ShapeDtypeStruct((E, EH), f32),
    )(edge_attr, W, b.reshape(1, EH))


def _proj(h, p):
    BN = 400

    def body(h_ref, wq, bq, wk, bk, wv, bv, ws, bs, we,
             q_ref, qe_ref, k_ref, v_ref, s_ref):
        hh = h_ref[...]
        q = jnp.dot(hh, wq[...], preferred_element_type=f32) + bq[...]
        q_ref[...] = q
        qe_ref[...] = lax.dot_general(q, we[...], (((1,), (1,)), ((), ())),
                                      preferred_element_type=f32)
        k_ref[...] = jnp.dot(hh, wk[...], preferred_element_type=f32) + bk[...]
        v_ref[...] = jnp.dot(hh, wv[...], preferred_element_type=f32) + bv[...]
        s_ref[...] = jnp.dot(hh, ws[...], preferred_element_type=f32) + bs[...]

    row = lambda i: (i, 0)
    fix = lambda i: (0, 0)
    return pl.pallas_call(
        body,
        grid=(N // BN,),
        in_specs=[pl.BlockSpec((BN, D), row),
                  pl.BlockSpec((D, D), fix), pl.BlockSpec((1, D), fix),
                  pl.BlockSpec((D, D), fix), pl.BlockSpec((1, D), fix),
                  pl.BlockSpec((D, D), fix), pl.BlockSpec((1, D), fix),
                  pl.BlockSpec((D, D), fix), pl.BlockSpec((1, D), fix),
                  pl.BlockSpec((EH, D), fix)],
        out_specs=[pl.BlockSpec((BN, D), row), pl.BlockSpec((BN, EH), row),
                   pl.BlockSpec((BN, D), row), pl.BlockSpec((BN, D), row),
                   pl.BlockSpec((BN, D), row)],
        out_shape=[jax.ShapeDtypeStruct((N, D), f32),
                   jax.ShapeDtypeStruct((N, EH), f32),
                   jax.ShapeDtypeStruct((N, D), f32),
                   jax.ShapeDtypeStruct((N, D), f32),
                   jax.ShapeDtypeStruct((N, D), f32)],
    )(h, p['Wq'], p['bq'].reshape(1, D), p['Wk'], p['bk'].reshape(1, D),
      p['Wv'], p['bv'].reshape(1, D), p['Ws'], p['bs'].reshape(1, D), p['We'])


def _combine(vp, ag, skip, we, relu):
    BN = 400

    def body(vp_ref, ag_ref, s_ref, we_ref, o_ref):
        h = vp_ref[0] + vp_ref[1] + s_ref[...]
        agg = ag_ref[0] + ag_ref[1]
        h = h + jnp.dot(agg, we_ref[...], preferred_element_type=f32)
        if relu:
            h = jnp.maximum(h, 0.0)
        o_ref[...] = h

    return pl.pallas_call(
        body,
        grid=(N // BN,),
        in_specs=[pl.BlockSpec((2, BN, D), lambda i: (0, i, 0)),
                  pl.BlockSpec((2, BN, EH), lambda i: (0, i, 0)),
                  pl.BlockSpec((BN, D), lambda i: (i, 0)),
                  pl.BlockSpec((EH, D), lambda i: (0, 0))],
        out_specs=pl.BlockSpec((BN, D), lambda i: (i, 0)),
        out_shape=jax.ShapeDtypeStruct((N, D), f32),
    )(vp, ag, skip, we)


@functools.partial(
    pl.kernel,
    out_type=[jax.ShapeDtypeStruct((E,), f32),
              jax.ShapeDtypeStruct((NC, NPAD), f32)],
    mesh=_mesh,
    compiler_params=pltpu.CompilerParams(needs_layout_passes=False, use_tc_tiling_on_sc=False),
    scratch_types=[
        pltpu.VMEM((CH,), i32),
        pltpu.VMEM((CH,), i32),
        pltpu.VMEM((CH, D), f32),
        pltpu.VMEM((CH, D), f32),
        pltpu.VMEM((CH, EH), f32),
        pltpu.VMEM((CH, EH), f32),
        pltpu.VMEM((CH,), i32),
        pltpu.VMEM((CH,), i32),
        pltpu.VMEM((CH, D), f32),
        pltpu.VMEM((CH, D), f32),
        pltpu.VMEM((CH, EH), f32),
        pltpu.VMEM((CH, EH), f32),
        pltpu.VMEM((CH,), f32),
        pltpu.VMEM((NPAD,), f32),
        pltpu.VMEM((SL,), f32),
        pltpu.VMEM((SL,), f32),
        pltpu.SemaphoreType.DMA,
        pltpu.SemaphoreType.DMA,
        pltpu.VMEM_SHARED((NS, NPAD), f32),
    ],
)
def _sc_alpha(q_hbm, qe_hbm, k_hbm, e_hbm, src_hbm, dst_hbm,
              alpha_hbm, mpart_hbm,
              idx_s0, idx_d0, qrows0, krows0, qerows0, erows0,
              idx_s1, idx_d1, qrows1, krows1, qerows1, erows1,
              abuf, mloc, acc, tmp, sem0, sem1, msh):
    c = lax.axis_index("c")
    s = lax.axis_index("s")
    wid = c * NS + s
    base = wid * EPW
    bufs = ((idx_s0, idx_d0, qrows0, krows0, qerows0, erows0, sem0),
            (idx_s1, idx_d1, qrows1, krows1, qerows1, erows1, sem1))

    def mi(i, carry):
        mloc[pl.ds(i * 16, 16)] = jnp.full((16,), NEG, f32)
        return carry
    lax.fori_loop(0, NPAD // 16, mi, 0)

    def issue(ci, B):
        iss, idd, qr, kr, qer, er, sem = B
        off = base + ci * CH
        pltpu.sync_copy(src_hbm.at[pl.ds(off, CH)], iss)
        pltpu.sync_copy(dst_hbm.at[pl.ds(off, CH)], idd)
        pltpu.sync_copy(e_hbm.at[pl.ds(off, CH)], er)
        pltpu.async_copy(k_hbm.at[iss], kr, sem)
        pltpu.async_copy(q_hbm.at[idd], qr, sem)
        pltpu.async_copy(qe_hbm.at[idd], qer, sem)

    def drain(B):
        iss, idd, qr, kr, qer, er, sem = B
        pltpu.make_async_copy(k_hbm.at[iss], kr, sem).wait()
        pltpu.make_async_copy(q_hbm.at[idd], qr, sem).wait()
        pltpu.make_async_copy(qe_hbm.at[idd], qer, sem).wait()

    def compute(ci, B):
        iss, idd, qrows, krows, qerows, erows, sem = B
        off = base + ci * CH
        lane = lax.broadcasted_iota(i32, (16,), 0)

        def grp(g, gcarry):
            def edge(j, accv):
                jj = g * 16 + j
                av = qrows[jj, pl.ds(0, 16)] * krows[jj, pl.ds(0, 16)]
                for t in range(1, D // 16):
                    av = av + (qrows[jj, pl.ds(16 * t, 16)]
                               * krows[jj, pl.ds(16 * t, 16)])
                av = av + qerows[jj, :] * erows[jj, :]
                for sh in (8, 4, 2, 1):
                    perm = jnp.bitwise_xor(lane, sh)
                    av = av + av.at[perm].get(mode='promise_in_bounds')
                return jnp.where(lane == j, av * SCALE, accv)
            a16 = lax.fori_loop(0, 16, edge, jnp.zeros((16,), f32))
            abuf[pl.ds(g * 16, 16)] = a16
            d16 = idd[pl.ds(g * 16, 16)]

            def bd(t, pend):
                msk = pend > 0
                old = plsc.load_gather(mloc, [d16])
                new = jnp.maximum(old, a16)
                plsc.store_scatter(mloc, [d16], new, mask=msk)
                chk = plsc.load_gather(mloc, [d16])
                return (msk & (chk < new)).astype(i32)
            lax.fori_loop(0, 16, bd, jnp.ones((16,), i32))
            return gcarry
        lax.fori_loop(0, CH // 16, grp, 0)
        pltpu.sync_copy(abuf, alpha_hbm.at[pl.ds(off, CH)])

    issue(0, bufs[0])

    def pair(pi, carry):
        for b in (0, 1):
            ci = 2 * pi + b
            issue(ci + 1, bufs[1 - b])
            drain(bufs[b])
            compute(ci, bufs[b])
        return carry
    lax.fori_loop(0, (NCHUNK - 1) // 2, pair, 0)
    drain(bufs[0])
    compute(NCHUNK - 1, bufs[0])

    pltpu.sync_copy(mloc, msh.at[s])
    plsc.subcore_barrier()
    col = s * SL
    pltpu.sync_copy(msh.at[0, pl.ds(col, SL)], acc)
    for t in range(1, NS):
        pltpu.sync_copy(msh.at[t, pl.ds(col, SL)], tmp)

        def mx(i, carry):
            acc[pl.ds(i * 16, 16)] = jnp.maximum(acc[pl.ds(i * 16, 16)],
                                                 tmp[pl.ds(i * 16, 16)])
            return carry
        lax.fori_loop(0, SL // 16, mx, 0)
    pltpu.sync_copy(acc, mpart_hbm.at[c, pl.ds(col, SL)])


@functools.partial(
    pl.kernel,
    out_type=[jax.ShapeDtypeStruct((E,), f32),
              jax.ShapeDtypeStruct((NC, NPAD), f32)],
    mesh=_mesh,
    compiler_params=pltpu.CompilerParams(needs_layout_passes=False, use_tc_tiling_on_sc=False),
    scratch_types=[
        pltpu.VMEM((NPAD,), f32),
        pltpu.VMEM((NPAD,), f32),
        pltpu.VMEM((CH,), i32),
        pltpu.VMEM((CH,), f32),
        pltpu.VMEM((CH,), f32),
        pltpu.VMEM((SL,), f32),
        pltpu.VMEM_SHARED((NPAD,), f32),
    ],
)
def _sc_exden(alpha_hbm, dst_hbm, mpart_hbm,
              ex_hbm, dpart_hbm,
              b0, b1, idx_d, abuf, exbuf, zb, dsh):
    c = lax.axis_index("c")
    s = lax.axis_index("s")
    wid = c * NS + s
    base = wid * EPW

    pltpu.sync_copy(mpart_hbm.at[0], b0)
    pltpu.sync_copy(mpart_hbm.at[1], b1)

    def mcomb(i, carry):
        m = jnp.maximum(b0[pl.ds(i * 16, 16)], b1[pl.ds(i * 16, 16)])
        fin = (m - m) == 0.0
        b0[pl.ds(i * 16, 16)] = jnp.where(fin, m, 0.0)
        return carry
    lax.fori_loop(0, NPAD // 16, mcomb, 0)

    def z(i, carry):
        zb[pl.ds(i * 16, 16)] = jnp.zeros((16,), f32)
        return carry
    lax.fori_loop(0, SL // 16, z, 0)
    col = s * SL
    pltpu.sync_copy(zb, dsh.at[pl.ds(col, SL)])
    plsc.subcore_barrier()

    def chunk(ci, carry):
        off = base + ci * CH
        pltpu.sync_copy(dst_hbm.at[pl.ds(off, CH)], idx_d)
        pltpu.sync_copy(alpha_hbm.at[pl.ds(off, CH)], abuf)

        def grp(g, gcarry):
            d16 = idx_d[pl.ds(g * 16, 16)]
            m16 = plsc.load_gather(b0, [d16])
            exbuf[pl.ds(g * 16, 16)] = jnp.exp(abuf[pl.ds(g * 16, 16)] - m16)
            return gcarry
        lax.fori_loop(0, CH // 16, grp, 0)
        pltpu.sync_copy(exbuf, ex_hbm.at[pl.ds(off, CH)])
        pltpu.sync_copy(exbuf, dsh.at[idx_d], add=True)
        return carry
    lax.fori_loop(0, NCHUNK, chunk, 0)

    plsc.subcore_barrier()
    pltpu.sync_copy(dsh.at[pl.ds(col, SL)], dpart_hbm.at[c, pl.ds(col, SL)])


@functools.partial(
    pl.kernel,
    out_type=[jax.ShapeDtypeStruct((E,), f32),
              jax.ShapeDtypeStruct((NC, N, EH), f32)],
    mesh=_mesh,
    compiler_params=pltpu.CompilerParams(needs_layout_passes=False, use_tc_tiling_on_sc=False),
    scratch_types=[
        pltpu.VMEM((NPAD,), f32),
        pltpu.VMEM((NPAD,), f32),
        pltpu.VMEM((CH,), i32),
        pltpu.VMEM((CH, EH), f32),
        pltpu.VMEM((CH,), f32),
        pltpu.VMEM((CH,), f32),
        pltpu.VMEM_SHARED((N, EH), f32),
    ],
)
def _sc_anorm(ex_hbm, dpart_hbm, e_hbm, dst_hbm,
              a_hbm, agpart_hbm,
              d0, d1, idx_d, erows, exbuf, abuf, agacc):
    c = lax.axis_index("c")
    s = lax.axis_index("s")
    wid = c * NS + s
    base = wid * EPW

    pltpu.sync_copy(dpart_hbm.at[0], d0)
    pltpu.sync_copy(dpart_hbm.at[1], d1)

    def dcomb(i, carry):
        d0[pl.ds(i * 16, 16)] = (d0[pl.ds(i * 16, 16)]
                                 + d1[pl.ds(i * 16, 16)] + 1e-16)
        return carry
    lax.fori_loop(0, NPAD // 16, dcomb, 0)

    def zrow(i, carry):
        erows[i, :] = jnp.zeros((16,), f32)
        return carry
    lax.fori_loop(0, CH, zrow, 0)

    row0 = s * NROW
    for (st, cnt) in ((0, 80), (80, 80), (160, 80), (240, 80),
                      (320, 80), (400, 80), (480, 80), (560, 65)):
        pltpu.sync_copy(erows.at[pl.ds(0, cnt)], agacc.at[pl.ds(row0 + st, cnt)])
    plsc.subcore_barrier()

    def chunk(ci, carry):
        off = base + ci * CH
        pltpu.sync_copy(dst_hbm.at[pl.ds(off, CH)], idx_d)
        pltpu.sync_copy(e_hbm.at[pl.ds(off, CH)], erows)
        pltpu.sync_copy(ex_hbm.at[pl.ds(off, CH)], exbuf)

        def grp(g, gcarry):
            d16 = idx_d[pl.ds(g * 16, 16)]
            den = plsc.load_gather(d0, [d16])
            a16 = exbuf[pl.ds(g * 16, 16)] / den
            abuf[pl.ds(g * 16, 16)] = a16

            def edge(l, ecarry):
                jj = g * 16 + l
                idx = lax.broadcast(l, (16,))
                aj = a16.at[idx].get(mode='promise_in_bounds')
                erows[jj, :] = erows[jj, :] * aj
                return ecarry
            lax.fori_loop(0, 16, edge, 0)
            return gcarry
        lax.fori_loop(0, CH // 16, grp, 0)

        pltpu.sync_copy(abuf, a_hbm.at[pl.ds(off, CH)])
        pltpu.sync_copy(erows, agacc.at[idx_d], add=True)
        return carry
    lax.fori_loop(0, NCHUNK, chunk, 0)

    plsc.subcore_barrier()
    pltpu.sync_copy(agacc.at[pl.ds(row0, NROW)],
                    agpart_hbm.at[c, pl.ds(row0, NROW)])


@functools.partial(
    pl.kernel,
    out_type=jax.ShapeDtypeStruct((NC, N, D), f32),
    mesh=_mesh,
    compiler_params=pltpu.CompilerParams(needs_layout_passes=False, use_tc_tiling_on_sc=False),
    scratch_types=[
        pltpu.VMEM((CH,), i32),
        pltpu.VMEM((CH,), i32),
        pltpu.VMEM((CH, D), f32),
        pltpu.VMEM((CH,), f32),
        pltpu.VMEM((CH,), i32),
        pltpu.VMEM((CH,), i32),
        pltpu.VMEM((CH, D), f32),
        pltpu.VMEM((CH,), f32),
        pltpu.SemaphoreType.DMA,
        pltpu.SemaphoreType.DMA,
        pltpu.VMEM_SHARED((N, D), f32),
    ],
)
def _sc_agg(a_hbm, v_hbm, src_hbm, dst_hbm,
            vpart_hbm,
            idx_s0, idx_d0, vrows0, abuf0,
            idx_s1, idx_d1, vrows1, abuf1,
            sem0, sem1, vacc):
    c = lax.axis_index("c")
    s = lax.axis_index("s")
    wid = c * NS + s
    base = wid * EPW
    bufs = ((idx_s0, idx_d0, vrows0, abuf0, sem0),
            (idx_s1, idx_d1, vrows1, abuf1, sem1))

    def zrow(i, carry):
        for t in range(D // 16):
            vrows0[i, pl.ds(16 * t, 16)] = jnp.zeros((16,), f32)
        return carry
    lax.fori_loop(0, CH, zrow, 0)

    row0 = s * NROW
    for (st, cnt) in ((0, 80), (80, 80), (160, 80), (240, 80),
                      (320, 80), (400, 80), (480, 80), (560, 65)):
        pltpu.sync_copy(vrows0.at[pl.ds(0, cnt)], vacc.at[pl.ds(row0 + st, cnt)])
    plsc.subcore_barrier()

    def issue(ci, B):
        iss, idd, vr, ab, sem = B
        off = base + ci * CH
        pltpu.sync_copy(src_hbm.at[pl.ds(off, CH)], iss)
        pltpu.sync_copy(dst_hbm.at[pl.ds(off, CH)], idd)
        pltpu.sync_copy(a_hbm.at[pl.ds(off, CH)], ab)
        pltpu.async_copy(v_hbm.at[iss], vr, sem)

    def drain(B):
        iss, idd, vr, ab, sem = B
        pltpu.make_async_copy(v_hbm.at[iss], vr, sem).wait()

    def compute(ci, B):
        iss, idd, vrows, abuf, sem = B

        def grp(g, gcarry):
            a16 = abuf[pl.ds(g * 16, 16)]

            def edge(l, ecarry):
                jj = g * 16 + l
                idx = lax.broadcast(l, (16,))
                aj = a16.at[idx].get(mode='promise_in_bounds')
                for t in range(D // 16):
                    vrows[jj, pl.ds(16 * t, 16)] = (
                        vrows[jj, pl.ds(16 * t, 16)] * aj)
                return ecarry
            lax.fori_loop(0, 16, edge, 0)
            return gcarry
        lax.fori_loop(0, CH // 16, grp, 0)
        pltpu.sync_copy(vrows, vacc.at[idd], add=True)

    issue(0, bufs[0])

    def pair(pi, carry):
        for b in (0, 1):
            ci = 2 * pi + b
            issue(ci + 1, bufs[1 - b])
            drain(bufs[b])
            compute(ci, bufs[b])
        return carry
    lax.fori_loop(0, (NCHUNK - 1) // 2, pair, 0)
    drain(bufs[0])
    compute(NCHUNK - 1, bufs[0])

    plsc.subcore_barrier()
    pltpu.sync_copy(vacc.at[pl.ds(row0, NROW)],
                    vpart_hbm.at[c, pl.ds(row0, NROW)])


def kernel(x, edge_index, edge_attr, params):
    src = edge_index[0].astype(i32)
    dst = edge_index[1].astype(i32)
    e = _edge_embed(edge_attr, params['W_emb'], params['b_emb'])
    h = x
    n_layers = len(params['layers'])
    for li, p in enumerate(params['layers']):
        q, qe, k, v, skip = _proj(h, p)
        alpha, mpart = _sc_alpha(q, qe, k, e, src, dst)
        ex, dpart = _sc_exden(alpha, dst, mpart)
        a, agpart = _sc_anorm(ex, dpart, e, dst)
        vpart = _sc_agg(a, v, src, dst)
        h = _combine(vpart, agpart, skip, p['We'], li < n_layers - 1)
    return h


# speedup vs baseline: 6.9203x; 1.0844x over previous
import functools

import jax
import jax.numpy as jnp
import numpy as np
from jax import lax
from jax.experimental import pallas as pl
from jax.experimental.pallas import tpu as pltpu
from jax.experimental.pallas import tpu_sc as plsc

N = 10000
E = 320000
D = 128
DE = 16
EH = 16

NC = 2
NS = 16
NW = NC * NS
EPW = E // NW
CH = 80
NCHUNK = EPW // CH
NPAD = 10240
SL = NPAD // NS
NROW = N // NS
SCALE = 1.0 / float(np.sqrt(D))
NEG = float("-inf")

_mesh = plsc.VectorSubcoreMesh(
    core_axis_name="c", subcore_axis_name="s", num_cores=NC, num_subcores=NS)

f32 = jnp.float32
i32 = jnp.int32


def _edge_embed(edge_attr, W, b):
    BE = 2000

    def body(ea_ref, w_ref, b_ref, o_ref):
        o_ref[...] = jnp.dot(ea_ref[...], w_ref[...],
                             preferred_element_type=f32) + b_ref[...]

    return pl.pallas_call(
        body,
        grid=(E // BE,),
        in_specs=[pl.BlockSpec((BE, DE), lambda i: (i, 0)),
                  pl.BlockSpec((DE, EH), lambda i: (0, 0)),
                  pl.BlockSpec((1, EH), lambda i: (0, 0))],
        out_specs=pl.BlockSpec((BE, EH), lambda i: (i, 0)),
        out_shape=jax.ShapeDtypeStruct((E, EH), f32),
    )(edge_attr, W, b.reshape(1, EH))


def _proj(h, p):
    BN = 400

    def body(h_ref, wq, bq, wk, bk, wv, bv, ws, bs, we,
             q_ref, qe_ref, k_ref, v_ref, s_ref):
        hh = h_ref[...]
        q = jnp.dot(hh, wq[...], preferred_element_type=f32) + bq[...]
        q_ref[...] = q
        qe_ref[...] = lax.dot_general(q, we[...], (((1,), (1,)), ((), ())),
                                      preferred_element_type=f32)
        k_ref[...] = jnp.dot(hh, wk[...], preferred_element_type=f32) + bk[...]
        v_ref[...] = jnp.dot(hh, wv[...], preferred_element_type=f32) + bv[...]
        s_ref[...] = jnp.dot(hh, ws[...], preferred_element_type=f32) + bs[...]

    row = lambda i: (i, 0)
    fix = lambda i: (0, 0)
    return pl.pallas_call(
        body,
        grid=(N // BN,),
        in_specs=[pl.BlockSpec((BN, D), row),
                  pl.BlockSpec((D, D), fix), pl.BlockSpec((1, D), fix),
                  pl.BlockSpec((D, D), fix), pl.BlockSpec((1, D), fix),
                  pl.BlockSpec((D, D), fix), pl.BlockSpec((1, D), fix),
                  pl.BlockSpec((D, D), fix), pl.BlockSpec((1, D), fix),
                  pl.BlockSpec((EH, D), fix)],
        out_specs=[pl.BlockSpec((BN, D), row), pl.BlockSpec((BN, EH), row),
                   pl.BlockSpec((BN, D), row), pl.BlockSpec((BN, D), row),
                   pl.BlockSpec((BN, D), row)],
        out_shape=[jax.ShapeDtypeStruct((N, D), f32),
                   jax.ShapeDtypeStruct((N, EH), f32),
                   jax.ShapeDtypeStruct((N, D), f32),
                   jax.ShapeDtypeStruct((N, D), f32),
                   jax.ShapeDtypeStruct((N, D), f32)],
    )(h, p['Wq'], p['bq'].reshape(1, D), p['Wk'], p['bk'].reshape(1, D),
      p['Wv'], p['bv'].reshape(1, D), p['Ws'], p['bs'].reshape(1, D), p['We'])


def _combine(vp, ag, skip, we, relu):
    BN = 400

    def body(vp_ref, ag_ref, s_ref, we_ref, o_ref):
        h = vp_ref[0] + vp_ref[1] + s_ref[...]
        agg = ag_ref[0] + ag_ref[1]
        h = h + jnp.dot(agg, we_ref[...], preferred_element_type=f32)
        if relu:
            h = jnp.maximum(h, 0.0)
        o_ref[...] = h

    return pl.pallas_call(
        body,
        grid=(N // BN,),
        in_specs=[pl.BlockSpec((2, BN, D), lambda i: (0, i, 0)),
                  pl.BlockSpec((2, BN, EH), lambda i: (0, i, 0)),
                  pl.BlockSpec((BN, D), lambda i: (i, 0)),
                  pl.BlockSpec((EH, D), lambda i: (0, 0))],
        out_specs=pl.BlockSpec((BN, D), lambda i: (i, 0)),
        out_shape=jax.ShapeDtypeStruct((N, D), f32),
    )(vp, ag, skip, we)


@functools.partial(
    pl.kernel,
    out_type=[jax.ShapeDtypeStruct((E,), f32),
              jax.ShapeDtypeStruct((NC, NPAD), f32)],
    mesh=_mesh,
    compiler_params=pltpu.CompilerParams(needs_layout_passes=False, use_tc_tiling_on_sc=False),
    scratch_types=[
        pltpu.VMEM((CH,), i32),
        pltpu.VMEM((CH,), i32),
        pltpu.VMEM((CH, D), f32),
        pltpu.VMEM((CH, D), f32),
        pltpu.VMEM((CH, EH), f32),
        pltpu.VMEM((CH, EH), f32),
        pltpu.VMEM((CH,), i32),
        pltpu.VMEM((CH,), i32),
        pltpu.VMEM((CH, D), f32),
        pltpu.VMEM((CH, D), f32),
        pltpu.VMEM((CH, EH), f32),
        pltpu.VMEM((CH, EH), f32),
        pltpu.VMEM((CH,), i32),
        pltpu.VMEM((CH,), i32),
        pltpu.VMEM((CH, D), f32),
        pltpu.VMEM((CH, D), f32),
        pltpu.VMEM((CH, EH), f32),
        pltpu.VMEM((CH, EH), f32),
        pltpu.VMEM((CH,), f32),
        pltpu.VMEM((NPAD,), f32),
        pltpu.VMEM((SL,), f32),
        pltpu.VMEM((SL,), f32),
        pltpu.SemaphoreType.DMA,
        pltpu.SemaphoreType.DMA,
        pltpu.SemaphoreType.DMA,
        pltpu.VMEM_SHARED((NS, NPAD), f32),
    ],
)
def _sc_alpha(q_hbm, qe_hbm, k_hbm, e_hbm, src_hbm, dst_hbm,
              alpha_hbm, mpart_hbm,
              idx_s0, idx_d0, qrows0, krows0, qerows0, erows0,
              idx_s1, idx_d1, qrows1, krows1, qerows1, erows1,
              idx_s2, idx_d2, qrows2, krows2, qerows2, erows2,
              abuf, mloc, acc, tmp, sem0, sem1, sem2, msh):
    c = lax.axis_index("c")
    s = lax.axis_index("s")
    wid = c * NS + s
    base = wid * EPW
    bufs = ((idx_s0, idx_d0, qrows0, krows0, qerows0, erows0, sem0),
            (idx_s1, idx_d1, qrows1, krows1, qerows1, erows1, sem1),
            (idx_s2, idx_d2, qrows2, krows2, qerows2, erows2, sem2))

    def mi(i, carry):
        mloc[pl.ds(i * 16, 16)] = jnp.full((16,), NEG, f32)
        return carry
    lax.fori_loop(0, NPAD // 16, mi, 0)

    def issue(ci, B):
        iss, idd, qr, kr, qer, er, sem = B
        off = base + ci * CH
        pltpu.sync_copy(src_hbm.at[pl.ds(off, CH)], iss)
        pltpu.sync_copy(dst_hbm.at[pl.ds(off, CH)], idd)
        pltpu.sync_copy(e_hbm.at[pl.ds(off, CH)], er)
        pltpu.async_copy(k_hbm.at[iss], kr, sem)
        pltpu.async_copy(q_hbm.at[idd], qr, sem)
        pltpu.async_copy(qe_hbm.at[idd], qer, sem)

    def drain(B):
        iss, idd, qr, kr, qer, er, sem = B
        pltpu.make_async_copy(k_hbm.at[iss], kr, sem).wait()
        pltpu.make_async_copy(q_hbm.at[idd], qr, sem).wait()
        pltpu.make_async_copy(qe_hbm.at[idd], qer, sem).wait()

    def compute(ci, B):
        iss, idd, qrows, krows, qerows, erows, sem = B
        off = base + ci * CH
        lane = lax.broadcasted_iota(i32, (16,), 0)

        def grp(g, gcarry):
            def edge(j, accv):
                jj = g * 16 + j
                av = qrows[jj, pl.ds(0, 16)] * krows[jj, pl.ds(0, 16)]
                for t in range(1, D // 16):
                    av = av + (qrows[jj, pl.ds(16 * t, 16)]
                               * krows[jj, pl.ds(16 * t, 16)])
                av = av + qerows[jj, :] * erows[jj, :]
                for sh in (8, 4, 2, 1):
                    perm = jnp.bitwise_xor(lane, sh)
                    av = av + av.at[perm].get(mode='promise_in_bounds')
                return jnp.where(lane == j, av * SCALE, accv)
            a16 = lax.fori_loop(0, 16, edge, jnp.zeros((16,), f32))
            abuf[pl.ds(g * 16, 16)] = a16
            d16 = idd[pl.ds(g * 16, 16)]

            old = plsc.load_gather(mloc, [d16])
            new = jnp.maximum(old, a16)
            plsc.store_scatter(mloc, [d16], new)
            chk = plsc.load_gather(mloc, [d16])
            pend0 = chk < new

            @pl.when(jnp.any(pend0))
            def _cleanup():
                def bd(t, pend):
                    msk = pend > 0
                    o = plsc.load_gather(mloc, [d16])
                    n = jnp.maximum(o, a16)
                    plsc.store_scatter(mloc, [d16], n, mask=msk)
                    k2 = plsc.load_gather(mloc, [d16])
                    return (msk & (k2 < n)).astype(i32)
                lax.fori_loop(0, 15, bd, pend0.astype(i32))
            return gcarry
        lax.fori_loop(0, CH // 16, grp, 0)
        pltpu.sync_copy(abuf, alpha_hbm.at[pl.ds(off, CH)])

    issue(0, bufs[0])
    issue(1, bufs[1])

    def tri(pi, carry):
        for b in (0, 1, 2):
            ci = 3 * pi + b
            issue(ci + 2, bufs[(b + 2) % 3])
            drain(bufs[b])
            compute(ci, bufs[b])
        return carry
    lax.fori_loop(0, (NCHUNK - 2) // 3, tri, 0)
    drain(bufs[0])
    compute(NCHUNK - 2, bufs[0])
    drain(bufs[1])
    compute(NCHUNK - 1, bufs[1])

    pltpu.sync_copy(mloc, msh.at[s])
    plsc.subcore_barrier()
    col = s * SL
    pltpu.sync_copy(msh.at[0, pl.ds(col, SL)], acc)
    for t in range(1, NS):
        pltpu.sync_copy(msh.at[t, pl.ds(col, SL)], tmp)

        def mx(i, carry):
            acc[pl.ds(i * 16, 16)] = jnp.maximum(acc[pl.ds(i * 16, 16)],
                                                 tmp[pl.ds(i * 16, 16)])
            return carry
        lax.fori_loop(0, SL // 16, mx, 0)
    pltpu.sync_copy(acc, mpart_hbm.at[c, pl.ds(col, SL)])


@functools.partial(
    pl.kernel,
    out_type=[jax.ShapeDtypeStruct((E,), f32),
              jax.ShapeDtypeStruct((NC, NPAD), f32)],
    mesh=_mesh,
    compiler_params=pltpu.CompilerParams(needs_layout_passes=False, use_tc_tiling_on_sc=False),
    scratch_types=[
        pltpu.VMEM((NPAD,), f32),
        pltpu.VMEM((NPAD,), f32),
        pltpu.VMEM((CH,), i32),
        pltpu.VMEM((CH,), f32),
        pltpu.VMEM((CH,), f32),
        pltpu.VMEM((SL,), f32),
        pltpu.VMEM_SHARED((NPAD,), f32),
    ],
)
def _sc_exden(alpha_hbm, dst_hbm, mpart_hbm,
              ex_hbm, dpart_hbm,
              b0, b1, idx_d, abuf, exbuf, zb, dsh):
    c = lax.axis_index("c")
    s = lax.axis_index("s")
    wid = c * NS + s
    base = wid * EPW

    pltpu.sync_copy(mpart_hbm.at[0], b0)
    pltpu.sync_copy(mpart_hbm.at[1], b1)

    def mcomb(i, carry):
        m = jnp.maximum(b0[pl.ds(i * 16, 16)], b1[pl.ds(i * 16, 16)])
        fin = (m - m) == 0.0
        b0[pl.ds(i * 16, 16)] = jnp.where(fin, m, 0.0)
        return carry
    lax.fori_loop(0, NPAD // 16, mcomb, 0)

    def z(i, carry):
        zb[pl.ds(i * 16, 16)] = jnp.zeros((16,), f32)
        return carry
    lax.fori_loop(0, SL // 16, z, 0)
    col = s * SL
    pltpu.sync_copy(zb, dsh.at[pl.ds(col, SL)])
    plsc.subcore_barrier()

    def chunk(ci, carry):
        off = base + ci * CH
        pltpu.sync_copy(dst_hbm.at[pl.ds(off, CH)], idx_d)
        pltpu.sync_copy(alpha_hbm.at[pl.ds(off, CH)], abuf)

        def grp(g, gcarry):
            d16 = idx_d[pl.ds(g * 16, 16)]
            m16 = plsc.load_gather(b0, [d16])
            exbuf[pl.ds(g * 16, 16)] = jnp.exp(abuf[pl.ds(g * 16, 16)] - m16)
            return gcarry
        lax.fori_loop(0, CH // 16, grp, 0)
        pltpu.sync_copy(exbuf, ex_hbm.at[pl.ds(off, CH)])
        pltpu.sync_copy(exbuf, dsh.at[idx_d], add=True)
        return carry
    lax.fori_loop(0, NCHUNK, chunk, 0)

    plsc.subcore_barrier()
    pltpu.sync_copy(dsh.at[pl.ds(col, SL)], dpart_hbm.at[c, pl.ds(col, SL)])


@functools.partial(
    pl.kernel,
    out_type=[jax.ShapeDtypeStruct((E,), f32),
              jax.ShapeDtypeStruct((NC, N, EH), f32)],
    mesh=_mesh,
    compiler_params=pltpu.CompilerParams(needs_layout_passes=False, use_tc_tiling_on_sc=False),
    scratch_types=[
        pltpu.VMEM((NPAD,), f32),
        pltpu.VMEM((NPAD,), f32),
        pltpu.VMEM((CH,), i32),
        pltpu.VMEM((CH, EH), f32),
        pltpu.VMEM((CH,), f32),
        pltpu.VMEM((CH,), f32),
        pltpu.VMEM_SHARED((N, EH), f32),
    ],
)
def _sc_anorm(ex_hbm, dpart_hbm, e_hbm, dst_hbm,
              a_hbm, agpart_hbm,
              d0, d1, idx_d, erows, exbuf, abuf, agacc):
    c = lax.axis_index("c")
    s = lax.axis_index("s")
    wid = c * NS + s
    base = wid * EPW

    pltpu.sync_copy(dpart_hbm.at[0], d0)
    pltpu.sync_copy(dpart_hbm.at[1], d1)

    def dcomb(i, carry):
        d0[pl.ds(i * 16, 16)] = (d0[pl.ds(i * 16, 16)]
                                 + d1[pl.ds(i * 16, 16)] + 1e-16)
        return carry
    lax.fori_loop(0, NPAD // 16, dcomb, 0)

    def zrow(i, carry):
        erows[i, :] = jnp.zeros((16,), f32)
        return carry
    lax.fori_loop(0, CH, zrow, 0)

    row0 = s * NROW
    for (st, cnt) in ((0, 80), (80, 80), (160, 80), (240, 80),
                      (320, 80), (400, 80), (480, 80), (560, 65)):
        pltpu.sync_copy(erows.at[pl.ds(0, cnt)], agacc.at[pl.ds(row0 + st, cnt)])
    plsc.subcore_barrier()

    def chunk(ci, carry):
        off = base + ci * CH
        pltpu.sync_copy(dst_hbm.at[pl.ds(off, CH)], idx_d)
        pltpu.sync_copy(e_hbm.at[pl.ds(off, CH)], erows)
        pltpu.sync_copy(ex_hbm.at[pl.ds(off, CH)], exbuf)

        def grp(g, gcarry):
            d16 = idx_d[pl.ds(g * 16, 16)]
            den = plsc.load_gather(d0, [d16])
            a16 = exbuf[pl.ds(g * 16, 16)] / den
            abuf[pl.ds(g * 16, 16)] = a16

            def edge(l, ecarry):
                jj = g * 16 + l
                idx = lax.broadcast(l, (16,))
                aj = a16.at[idx].get(mode='promise_in_bounds')
                erows[jj, :] = erows[jj, :] * aj
                return ecarry
            lax.fori_loop(0, 16, edge, 0)
            return gcarry
        lax.fori_loop(0, CH // 16, grp, 0)

        pltpu.sync_copy(abuf, a_hbm.at[pl.ds(off, CH)])
        pltpu.sync_copy(erows, agacc.at[idx_d], add=True)
        return carry
    lax.fori_loop(0, NCHUNK, chunk, 0)

    plsc.subcore_barrier()
    pltpu.sync_copy(agacc.at[pl.ds(row0, NROW)],
                    agpart_hbm.at[c, pl.ds(row0, NROW)])


@functools.partial(
    pl.kernel,
    out_type=jax.ShapeDtypeStruct((NC, N, D), f32),
    mesh=_mesh,
    compiler_params=pltpu.CompilerParams(needs_layout_passes=False, use_tc_tiling_on_sc=False),
    scratch_types=[
        pltpu.VMEM((CH,), i32),
        pltpu.VMEM((CH,), i32),
        pltpu.VMEM((CH, D), f32),
        pltpu.VMEM((CH,), f32),
        pltpu.VMEM((CH,), i32),
        pltpu.VMEM((CH,), i32),
        pltpu.VMEM((CH, D), f32),
        pltpu.VMEM((CH,), f32),
        pltpu.VMEM((CH,), i32),
        pltpu.VMEM((CH,), i32),
        pltpu.VMEM((CH, D), f32),
        pltpu.VMEM((CH,), f32),
        pltpu.SemaphoreType.DMA,
        pltpu.SemaphoreType.DMA,
        pltpu.SemaphoreType.DMA,
        pltpu.SemaphoreType.DMA,
        pltpu.SemaphoreType.DMA,
        pltpu.SemaphoreType.DMA,
        pltpu.VMEM_SHARED((N, D), f32),
    ],
)
def _sc_agg(a_hbm, v_hbm, src_hbm, dst_hbm,
            vpart_hbm,
            idx_s0, idx_d0, vrows0, abuf0,
            idx_s1, idx_d1, vrows1, abuf1,
            idx_s2, idx_d2, vrows2, abuf2,
            gsem0, gsem1, gsem2, ssem0, ssem1, ssem2, vacc):
    c = lax.axis_index("c")
    s = lax.axis_index("s")
    wid = c * NS + s
    base = wid * EPW
    bufs = ((idx_s0, idx_d0, vrows0, abuf0, gsem0, ssem0),
            (idx_s1, idx_d1, vrows1, abuf1, gsem1, ssem1),
            (idx_s2, idx_d2, vrows2, abuf2, gsem2, ssem2))

    def zrow(i, carry):
        for t in range(D // 16):
            vrows0[i, pl.ds(16 * t, 16)] = jnp.zeros((16,), f32)
        return carry
    lax.fori_loop(0, CH, zrow, 0)

    row0 = s * NROW
    for (st, cnt) in ((0, 80), (80, 80), (160, 80), (240, 80),
                      (320, 80), (400, 80), (480, 80), (560, 65)):
        pltpu.sync_copy(vrows0.at[pl.ds(0, cnt)], vacc.at[pl.ds(row0 + st, cnt)])
    plsc.subcore_barrier()

    def issue(ci, B, wait_scatter):
        iss, idd, vr, ab, gsem, ssem = B
        if wait_scatter:
            pltpu.make_async_copy(vr, vacc.at[idd], ssem).wait()
        off = base + ci * CH
        pltpu.sync_copy(src_hbm.at[pl.ds(off, CH)], iss)
        pltpu.sync_copy(dst_hbm.at[pl.ds(off, CH)], idd)
        pltpu.sync_copy(a_hbm.at[pl.ds(off, CH)], ab)
        pltpu.async_copy(v_hbm.at[iss], vr, gsem)

    def compute(ci, B):
        iss, idd, vrows, abuf, gsem, ssem = B
        pltpu.make_async_copy(v_hbm.at[iss], vrows, gsem).wait()

        def grp(g, gcarry):
            a16 = abuf[pl.ds(g * 16, 16)]

            def edge(l, ecarry):
                jj = g * 16 + l
                idx = lax.broadcast(l, (16,))
                aj = a16.at[idx].get(mode='promise_in_bounds')
                for t in range(D // 16):
                    vrows[jj, pl.ds(16 * t, 16)] = (
                        vrows[jj, pl.ds(16 * t, 16)] * aj)
                return ecarry
            lax.fori_loop(0, 16, edge, 0)
            return gcarry
        lax.fori_loop(0, CH // 16, grp, 0)
        pltpu.async_copy(vrows, vacc.at[idd], ssem, add=True)

    issue(0, bufs[0], False)
    issue(1, bufs[1], False)
    compute(0, bufs[0])
    issue(2, bufs[2], False)
    compute(1, bufs[1])
    issue(3, bufs[0], True)
    compute(2, bufs[2])
    issue(4, bufs[1], True)

    def tri(pi, carry):
        for b in (0, 1, 2):
            ci = 3 * pi + b
            compute(ci, bufs[b])
            issue(ci + 2, bufs[(b + 2) % 3], True)
        return carry
    lax.fori_loop(1, (NCHUNK - 2) // 3, tri, 0)
    compute(NCHUNK - 2, bufs[0])
    compute(NCHUNK - 1, bufs[1])
    for b in (2, 0, 1):
        iss, idd, vr, ab, gsem, ssem = bufs[b]
        pltpu.make_async_copy(vr, vacc.at[idd], ssem).wait()
    plsc.subcore_barrier()
    pltpu.sync_copy(vacc.at[pl.ds(row0, NROW)],
                    vpart_hbm.at[c, pl.ds(row0, NROW)])


def kernel(x, edge_index, edge_attr, params):
    src = edge_index[0].astype(i32)
    dst = edge_index[1].astype(i32)
    e = _edge_embed(edge_attr, params['W_emb'], params['b_emb'])
    h = x
    n_layers = len(params['layers'])
    for li, p in enumerate(params['layers']):
        q, qe, k, v, skip = _proj(h, p)
        alpha, mpart = _sc_alpha(q, qe, k, e, src, dst)
        ex, dpart = _sc_exden(alpha, dst, mpart)
        a, agpart = _sc_anorm(ex, dpart, e, dst)
        vpart = _sc_agg(a, v, src, dst)
        h = _combine(vpart, agpart, skip, p['We'], li < n_layers - 1)
    return h


# speedup vs baseline: 7.5060x; 1.0846x over previous
import functools

import jax
import jax.numpy as jnp
import numpy as np
from jax import lax
from jax.experimental import pallas as pl
from jax.experimental.pallas import tpu as pltpu
from jax.experimental.pallas import tpu_sc as plsc

N = 10000
E = 320000
D = 128
DE = 16
EH = 16

NC = 2
NS = 16
NW = NC * NS
EPW = E // NW
CH = 80
NCHUNK = EPW // CH
NPAD = 10240
SL = NPAD // NS
NROW = N // NS
SCALE = 1.0 / float(np.sqrt(D))
NEG = float("-inf")

_mesh = plsc.VectorSubcoreMesh(
    core_axis_name="c", subcore_axis_name="s", num_cores=NC, num_subcores=NS)

f32 = jnp.float32
i32 = jnp.int32


def _edge_embed(edge_attr, W, b):
    BE = 2000

    def body(ea_ref, w_ref, b_ref, o_ref):
        o_ref[...] = jnp.dot(ea_ref[...], w_ref[...],
                             preferred_element_type=f32) + b_ref[...]

    return pl.pallas_call(
        body,
        grid=(E // BE,),
        in_specs=[pl.BlockSpec((BE, DE), lambda i: (i, 0)),
                  pl.BlockSpec((DE, EH), lambda i: (0, 0)),
                  pl.BlockSpec((1, EH), lambda i: (0, 0))],
        out_specs=pl.BlockSpec((BE, EH), lambda i: (i, 0)),
        out_shape=jax.ShapeDtypeStruct((E, EH), f32),
    )(edge_attr, W, b.reshape(1, EH))


def _proj(h, p):
    BN = 400

    def body(h_ref, wq, bq, wk, bk, wv, bv, ws, bs, we,
             q_ref, qe_ref, k_ref, v_ref, s_ref):
        hh = h_ref[...]
        q = jnp.dot(hh, wq[...], preferred_element_type=f32) + bq[...]
        q_ref[...] = q
        qe_ref[...] = lax.dot_general(q, we[...], (((1,), (1,)), ((), ())),
                                      preferred_element_type=f32)
        k_ref[...] = jnp.dot(hh, wk[...], preferred_element_type=f32) + bk[...]
        v_ref[...] = jnp.dot(hh, wv[...], preferred_element_type=f32) + bv[...]
        s_ref[...] = jnp.dot(hh, ws[...], preferred_element_type=f32) + bs[...]

    row = lambda i: (i, 0)
    fix = lambda i: (0, 0)
    return pl.pallas_call(
        body,
        grid=(N // BN,),
        in_specs=[pl.BlockSpec((BN, D), row),
                  pl.BlockSpec((D, D), fix), pl.BlockSpec((1, D), fix),
                  pl.BlockSpec((D, D), fix), pl.BlockSpec((1, D), fix),
                  pl.BlockSpec((D, D), fix), pl.BlockSpec((1, D), fix),
                  pl.BlockSpec((D, D), fix), pl.BlockSpec((1, D), fix),
                  pl.BlockSpec((EH, D), fix)],
        out_specs=[pl.BlockSpec((BN, D), row), pl.BlockSpec((BN, EH), row),
                   pl.BlockSpec((BN, D), row), pl.BlockSpec((BN, D), row),
                   pl.BlockSpec((BN, D), row)],
        out_shape=[jax.ShapeDtypeStruct((N, D), f32),
                   jax.ShapeDtypeStruct((N, EH), f32),
                   jax.ShapeDtypeStruct((N, D), f32),
                   jax.ShapeDtypeStruct((N, D), f32),
                   jax.ShapeDtypeStruct((N, D), f32)],
    )(h, p['Wq'], p['bq'].reshape(1, D), p['Wk'], p['bk'].reshape(1, D),
      p['Wv'], p['bv'].reshape(1, D), p['Ws'], p['bs'].reshape(1, D), p['We'])


def _combine(vp, ag, skip, we, relu):
    BN = 400

    def body(vp_ref, ag_ref, s_ref, we_ref, o_ref):
        h = vp_ref[0] + vp_ref[1] + s_ref[...]
        agg = ag_ref[0] + ag_ref[1]
        h = h + jnp.dot(agg, we_ref[...], preferred_element_type=f32)
        if relu:
            h = jnp.maximum(h, 0.0)
        o_ref[...] = h

    return pl.pallas_call(
        body,
        grid=(N // BN,),
        in_specs=[pl.BlockSpec((2, BN, D), lambda i: (0, i, 0)),
                  pl.BlockSpec((2, BN, EH), lambda i: (0, i, 0)),
                  pl.BlockSpec((BN, D), lambda i: (i, 0)),
                  pl.BlockSpec((EH, D), lambda i: (0, 0))],
        out_specs=pl.BlockSpec((BN, D), lambda i: (i, 0)),
        out_shape=jax.ShapeDtypeStruct((N, D), f32),
    )(vp, ag, skip, we)


@functools.partial(
    pl.kernel,
    out_type=[jax.ShapeDtypeStruct((E,), f32),
              jax.ShapeDtypeStruct((NC, NPAD), f32)],
    mesh=_mesh,
    compiler_params=pltpu.CompilerParams(needs_layout_passes=False, use_tc_tiling_on_sc=False),
    scratch_types=[
        pltpu.VMEM((CH,), i32),
        pltpu.VMEM((CH,), i32),
        pltpu.VMEM((CH, D), f32),
        pltpu.VMEM((CH, D), f32),
        pltpu.VMEM((CH, EH), f32),
        pltpu.VMEM((CH, EH), f32),
        pltpu.VMEM((CH,), i32),
        pltpu.VMEM((CH,), i32),
        pltpu.VMEM((CH, D), f32),
        pltpu.VMEM((CH, D), f32),
        pltpu.VMEM((CH, EH), f32),
        pltpu.VMEM((CH, EH), f32),
        pltpu.VMEM((CH,), i32),
        pltpu.VMEM((CH,), i32),
        pltpu.VMEM((CH, D), f32),
        pltpu.VMEM((CH, D), f32),
        pltpu.VMEM((CH, EH), f32),
        pltpu.VMEM((CH, EH), f32),
        pltpu.VMEM((CH,), f32),
        pltpu.VMEM((NPAD,), f32),
        pltpu.VMEM((SL,), f32),
        pltpu.VMEM((SL,), f32),
        pltpu.SemaphoreType.DMA,
        pltpu.SemaphoreType.DMA,
        pltpu.SemaphoreType.DMA,
        pltpu.VMEM_SHARED((NS, NPAD), f32),
    ],
)
def _sc_alpha(q_hbm, qe_hbm, k_hbm, e_hbm, src_hbm, dst_hbm,
              alpha_hbm, mpart_hbm,
              idx_s0, idx_d0, qrows0, krows0, qerows0, erows0,
              idx_s1, idx_d1, qrows1, krows1, qerows1, erows1,
              idx_s2, idx_d2, qrows2, krows2, qerows2, erows2,
              abuf, mloc, acc, tmp, sem0, sem1, sem2, msh):
    c = lax.axis_index("c")
    s = lax.axis_index("s")
    wid = c * NS + s
    base = wid * EPW
    bufs = ((idx_s0, idx_d0, qrows0, krows0, qerows0, erows0, sem0),
            (idx_s1, idx_d1, qrows1, krows1, qerows1, erows1, sem1),
            (idx_s2, idx_d2, qrows2, krows2, qerows2, erows2, sem2))

    def mi(i, carry):
        mloc[pl.ds(i * 16, 16)] = jnp.full((16,), NEG, f32)
        return carry
    lax.fori_loop(0, NPAD // 16, mi, 0)

    def issue(ci, B):
        iss, idd, qr, kr, qer, er, sem = B
        off = base + ci * CH
        pltpu.sync_copy(src_hbm.at[pl.ds(off, CH)], iss)
        pltpu.sync_copy(dst_hbm.at[pl.ds(off, CH)], idd)
        pltpu.sync_copy(e_hbm.at[pl.ds(off, CH)], er)
        pltpu.async_copy(k_hbm.at[iss], kr, sem)
        pltpu.async_copy(q_hbm.at[idd], qr, sem)
        pltpu.async_copy(qe_hbm.at[idd], qer, sem)

    def drain(B):
        iss, idd, qr, kr, qer, er, sem = B
        pltpu.make_async_copy(k_hbm.at[iss], kr, sem).wait()
        pltpu.make_async_copy(q_hbm.at[idd], qr, sem).wait()
        pltpu.make_async_copy(qe_hbm.at[idd], qer, sem).wait()

    def compute(ci, B):
        iss, idd, qrows, krows, qerows, erows, sem = B
        off = base + ci * CH
        lane = lax.broadcasted_iota(i32, (16,), 0)

        def grp(g, gcarry):
            def edge(j, accv):
                jj = g * 16 + j
                av = qrows[jj, pl.ds(0, 16)] * krows[jj, pl.ds(0, 16)]
                for t in range(1, D // 16):
                    av = av + (qrows[jj, pl.ds(16 * t, 16)]
                               * krows[jj, pl.ds(16 * t, 16)])
                av = av + qerows[jj, :] * erows[jj, :]
                for sh in (8, 4, 2, 1):
                    perm = jnp.bitwise_xor(lane, sh)
                    av = av + av.at[perm].get(mode='promise_in_bounds')
                return jnp.where(lane == j, av * SCALE, accv)
            a16 = lax.fori_loop(0, 16, edge, jnp.zeros((16,), f32))
            abuf[pl.ds(g * 16, 16)] = a16
            d16 = idd[pl.ds(g * 16, 16)]

            old = plsc.load_gather(mloc, [d16])
            new = jnp.maximum(old, a16)
            plsc.store_scatter(mloc, [d16], new)
            chk = plsc.load_gather(mloc, [d16])
            pend0 = chk < new

            @pl.when(jnp.any(pend0))
            def _cleanup():
                def bd(t, pend):
                    msk = pend > 0
                    o = plsc.load_gather(mloc, [d16])
                    n = jnp.maximum(o, a16)
                    plsc.store_scatter(mloc, [d16], n, mask=msk)
                    k2 = plsc.load_gather(mloc, [d16])
                    return (msk & (k2 < n)).astype(i32)
                lax.fori_loop(0, 15, bd, pend0.astype(i32))
            return gcarry
        lax.fori_loop(0, CH // 16, grp, 0)
        pltpu.sync_copy(abuf, alpha_hbm.at[pl.ds(off, CH)])

    issue(0, bufs[0])
    issue(1, bufs[1])

    def tri(pi, carry):
        for b in (0, 1, 2):
            ci = 3 * pi + b
            issue(ci + 2, bufs[(b + 2) % 3])
            drain(bufs[b])
            compute(ci, bufs[b])
        return carry
    lax.fori_loop(0, (NCHUNK - 2) // 3, tri, 0)
    drain(bufs[0])
    compute(NCHUNK - 2, bufs[0])
    drain(bufs[1])
    compute(NCHUNK - 1, bufs[1])

    pltpu.sync_copy(mloc, msh.at[s])
    plsc.subcore_barrier()
    col = s * SL
    pltpu.sync_copy(msh.at[0, pl.ds(col, SL)], acc)
    for t in range(1, NS):
        pltpu.sync_copy(msh.at[t, pl.ds(col, SL)], tmp)

        def mx(i, carry):
            acc[pl.ds(i * 16, 16)] = jnp.maximum(acc[pl.ds(i * 16, 16)],
                                                 tmp[pl.ds(i * 16, 16)])
            return carry
        lax.fori_loop(0, SL // 16, mx, 0)
    pltpu.sync_copy(acc, mpart_hbm.at[c, pl.ds(col, SL)])


@functools.partial(
    pl.kernel,
    out_type=[jax.ShapeDtypeStruct((E,), f32),
              jax.ShapeDtypeStruct((NC, NPAD), f32)],
    mesh=_mesh,
    compiler_params=pltpu.CompilerParams(needs_layout_passes=False, use_tc_tiling_on_sc=False),
    scratch_types=[
        pltpu.VMEM((NPAD,), f32),
        pltpu.VMEM((NPAD,), f32),
        pltpu.VMEM((CH,), i32),
        pltpu.VMEM((CH,), f32),
        pltpu.VMEM((CH,), f32),
        pltpu.VMEM((CH,), i32),
        pltpu.VMEM((CH,), f32),
        pltpu.VMEM((CH,), f32),
        pltpu.VMEM((CH,), i32),
        pltpu.VMEM((CH,), f32),
        pltpu.VMEM((CH,), f32),
        pltpu.VMEM((SL,), f32),
        pltpu.SemaphoreType.DMA,
        pltpu.SemaphoreType.DMA,
        pltpu.SemaphoreType.DMA,
        pltpu.VMEM_SHARED((NPAD,), f32),
    ],
)
def _sc_exden(alpha_hbm, dst_hbm, mpart_hbm,
              ex_hbm, dpart_hbm,
              b0, b1,
              idx_d0, abuf0, exbuf0,
              idx_d1, abuf1, exbuf1,
              idx_d2, abuf2, exbuf2,
              zb, ssem0, ssem1, ssem2, dsh):
    c = lax.axis_index("c")
    s = lax.axis_index("s")
    wid = c * NS + s
    base = wid * EPW
    bufs = ((idx_d0, abuf0, exbuf0, ssem0),
            (idx_d1, abuf1, exbuf1, ssem1),
            (idx_d2, abuf2, exbuf2, ssem2))

    pltpu.sync_copy(mpart_hbm.at[0], b0)
    pltpu.sync_copy(mpart_hbm.at[1], b1)

    def mcomb(i, carry):
        m = jnp.maximum(b0[pl.ds(i * 16, 16)], b1[pl.ds(i * 16, 16)])
        fin = (m - m) == 0.0
        b0[pl.ds(i * 16, 16)] = jnp.where(fin, m, 0.0)
        return carry
    lax.fori_loop(0, NPAD // 16, mcomb, 0)

    def z(i, carry):
        zb[pl.ds(i * 16, 16)] = jnp.zeros((16,), f32)
        return carry
    lax.fori_loop(0, SL // 16, z, 0)
    col = s * SL
    pltpu.sync_copy(zb, dsh.at[pl.ds(col, SL)])
    plsc.subcore_barrier()

    def work(ci, B, wait_scatter):
        idx_d, abuf, exbuf, ssem = B
        if wait_scatter:
            pltpu.make_async_copy(exbuf, dsh.at[idx_d], ssem).wait()
        off = base + ci * CH
        pltpu.sync_copy(dst_hbm.at[pl.ds(off, CH)], idx_d)
        pltpu.sync_copy(alpha_hbm.at[pl.ds(off, CH)], abuf)

        def grp(g, gcarry):
            d16 = idx_d[pl.ds(g * 16, 16)]
            m16 = plsc.load_gather(b0, [d16])
            exbuf[pl.ds(g * 16, 16)] = jnp.exp(abuf[pl.ds(g * 16, 16)] - m16)
            return gcarry
        lax.fori_loop(0, CH // 16, grp, 0)
        pltpu.sync_copy(exbuf, ex_hbm.at[pl.ds(off, CH)])
        pltpu.async_copy(exbuf, dsh.at[idx_d], ssem, add=True)

    work(0, bufs[0], False)
    work(1, bufs[1], False)
    work(2, bufs[2], False)

    def tri(pi, carry):
        for b in (0, 1, 2):
            work(3 * pi + b, bufs[b], True)
        return carry
    lax.fori_loop(1, NCHUNK // 3, tri, 0)
    work(NCHUNK - 2, bufs[0], True)
    work(NCHUNK - 1, bufs[1], True)
    for b in (2, 0, 1):
        idx_d, abuf, exbuf, ssem = bufs[b]
        pltpu.make_async_copy(exbuf, dsh.at[idx_d], ssem).wait()
    plsc.subcore_barrier()
    pltpu.sync_copy(dsh.at[pl.ds(col, SL)], dpart_hbm.at[c, pl.ds(col, SL)])


@functools.partial(
    pl.kernel,
    out_type=[jax.ShapeDtypeStruct((E,), f32),
              jax.ShapeDtypeStruct((NC, N, EH), f32)],
    mesh=_mesh,
    compiler_params=pltpu.CompilerParams(needs_layout_passes=False, use_tc_tiling_on_sc=False),
    scratch_types=[
        pltpu.VMEM((NPAD,), f32),
        pltpu.VMEM((NPAD,), f32),
        pltpu.VMEM((CH,), i32),
        pltpu.VMEM((CH, EH), f32),
        pltpu.VMEM((CH,), f32),
        pltpu.VMEM((CH,), f32),
        pltpu.VMEM((CH,), i32),
        pltpu.VMEM((CH, EH), f32),
        pltpu.VMEM((CH,), f32),
        pltpu.VMEM((CH,), f32),
        pltpu.VMEM((CH,), i32),
        pltpu.VMEM((CH, EH), f32),
        pltpu.VMEM((CH,), f32),
        pltpu.VMEM((CH,), f32),
        pltpu.SemaphoreType.DMA,
        pltpu.SemaphoreType.DMA,
        pltpu.SemaphoreType.DMA,
        pltpu.SemaphoreType.DMA,
        pltpu.SemaphoreType.DMA,
        pltpu.SemaphoreType.DMA,
        pltpu.VMEM_SHARED((N, EH), f32),
    ],
)
def _sc_anorm(ex_hbm, dpart_hbm, e_hbm, dst_hbm,
              a_hbm, agpart_hbm,
              d0, d1,
              idx_d0, erows0, exbuf0, abuf0,
              idx_d1, erows1, exbuf1, abuf1,
              idx_d2, erows2, exbuf2, abuf2,
              gsem0, gsem1, gsem2, ssem0, ssem1, ssem2, agacc):
    c = lax.axis_index("c")
    s = lax.axis_index("s")
    wid = c * NS + s
    base = wid * EPW
    bufs = ((idx_d0, erows0, exbuf0, abuf0, gsem0, ssem0),
            (idx_d1, erows1, exbuf1, abuf1, gsem1, ssem1),
            (idx_d2, erows2, exbuf2, abuf2, gsem2, ssem2))

    pltpu.sync_copy(dpart_hbm.at[0], d0)
    pltpu.sync_copy(dpart_hbm.at[1], d1)

    def dcomb(i, carry):
        d0[pl.ds(i * 16, 16)] = (d0[pl.ds(i * 16, 16)]
                                 + d1[pl.ds(i * 16, 16)] + 1e-16)
        return carry
    lax.fori_loop(0, NPAD // 16, dcomb, 0)

    def zrow(i, carry):
        erows0[i, :] = jnp.zeros((16,), f32)
        return carry
    lax.fori_loop(0, CH, zrow, 0)

    row0 = s * NROW
    for (st, cnt) in ((0, 80), (80, 80), (160, 80), (240, 80),
                      (320, 80), (400, 80), (480, 80), (560, 65)):
        pltpu.sync_copy(erows0.at[pl.ds(0, cnt)], agacc.at[pl.ds(row0 + st, cnt)])
    plsc.subcore_barrier()

    def issue(ci, B, wait_scatter):
        idd, er, exb, ab, gsem, ssem = B
        if wait_scatter:
            pltpu.make_async_copy(er, agacc.at[idd], ssem).wait()
        off = base + ci * CH
        pltpu.sync_copy(dst_hbm.at[pl.ds(off, CH)], idd)
        pltpu.sync_copy(ex_hbm.at[pl.ds(off, CH)], exb)
        pltpu.async_copy(e_hbm.at[pl.ds(off, CH)], er, gsem)

    def compute(ci, B):
        idd, erows, exbuf, abuf, gsem, ssem = B
        off = base + ci * CH
        pltpu.make_async_copy(e_hbm.at[pl.ds(off, CH)], erows, gsem).wait()

        def grp(g, gcarry):
            d16 = idd[pl.ds(g * 16, 16)]
            den = plsc.load_gather(d0, [d16])
            a16 = exbuf[pl.ds(g * 16, 16)] / den
            abuf[pl.ds(g * 16, 16)] = a16

            def edge(l, ecarry):
                jj = g * 16 + l
                idx = lax.broadcast(l, (16,))
                aj = a16.at[idx].get(mode='promise_in_bounds')
                erows[jj, :] = erows[jj, :] * aj
                return ecarry
            lax.fori_loop(0, 16, edge, 0)
            return gcarry
        lax.fori_loop(0, CH // 16, grp, 0)

        pltpu.sync_copy(abuf, a_hbm.at[pl.ds(off, CH)])
        pltpu.async_copy(erows, agacc.at[idd], ssem, add=True)

    issue(0, bufs[0], False)
    issue(1, bufs[1], False)
    compute(0, bufs[0])
    issue(2, bufs[2], False)
    compute(1, bufs[1])
    issue(3, bufs[0], True)
    compute(2, bufs[2])
    issue(4, bufs[1], True)

    def tri(pi, carry):
        for b in (0, 1, 2):
            ci = 3 * pi + b
            compute(ci, bufs[b])
            issue(ci + 2, bufs[(b + 2) % 3], True)
        return carry
    lax.fori_loop(1, (NCHUNK - 2) // 3, tri, 0)
    compute(NCHUNK - 2, bufs[0])
    compute(NCHUNK - 1, bufs[1])
    for b in (2, 0, 1):
        idd, er, exb, ab, gsem, ssem = bufs[b]
        pltpu.make_async_copy(er, agacc.at[idd], ssem).wait()
    plsc.subcore_barrier()
    pltpu.sync_copy(agacc.at[pl.ds(row0, NROW)],
                    agpart_hbm.at[c, pl.ds(row0, NROW)])


@functools.partial(
    pl.kernel,
    out_type=jax.ShapeDtypeStruct((NC, N, D), f32),
    mesh=_mesh,
    compiler_params=pltpu.CompilerParams(needs_layout_passes=False, use_tc_tiling_on_sc=False),
    scratch_types=[
        pltpu.VMEM((CH,), i32),
        pltpu.VMEM((CH,), i32),
        pltpu.VMEM((CH, D), f32),
        pltpu.VMEM((CH,), f32),
        pltpu.VMEM((CH,), i32),
        pltpu.VMEM((CH,), i32),
        pltpu.VMEM((CH, D), f32),
        pltpu.VMEM((CH,), f32),
        pltpu.VMEM((CH,), i32),
        pltpu.VMEM((CH,), i32),
        pltpu.VMEM((CH, D), f32),
        pltpu.VMEM((CH,), f32),
        pltpu.SemaphoreType.DMA,
        pltpu.SemaphoreType.DMA,
        pltpu.SemaphoreType.DMA,
        pltpu.SemaphoreType.DMA,
        pltpu.SemaphoreType.DMA,
        pltpu.SemaphoreType.DMA,
        pltpu.VMEM_SHARED((N, D), f32),
    ],
)
def _sc_agg(a_hbm, v_hbm, src_hbm, dst_hbm,
            vpart_hbm,
            idx_s0, idx_d0, vrows0, abuf0,
            idx_s1, idx_d1, vrows1, abuf1,
            idx_s2, idx_d2, vrows2, abuf2,
            gsem0, gsem1, gsem2, ssem0, ssem1, ssem2, vacc):
    c = lax.axis_index("c")
    s = lax.axis_index("s")
    wid = c * NS + s
    base = wid * EPW
    bufs = ((idx_s0, idx_d0, vrows0, abuf0, gsem0, ssem0),
            (idx_s1, idx_d1, vrows1, abuf1, gsem1, ssem1),
            (idx_s2, idx_d2, vrows2, abuf2, gsem2, ssem2))

    def zrow(i, carry):
        for t in range(D // 16):
            vrows0[i, pl.ds(16 * t, 16)] = jnp.zeros((16,), f32)
        return carry
    lax.fori_loop(0, CH, zrow, 0)

    row0 = s * NROW
    for (st, cnt) in ((0, 80), (80, 80), (160, 80), (240, 80),
                      (320, 80), (400, 80), (480, 80), (560, 65)):
        pltpu.sync_copy(vrows0.at[pl.ds(0, cnt)], vacc.at[pl.ds(row0 + st, cnt)])
    plsc.subcore_barrier()

    def issue(ci, B, wait_scatter):
        iss, idd, vr, ab, gsem, ssem = B
        if wait_scatter:
            pltpu.make_async_copy(vr, vacc.at[idd], ssem).wait()
        off = base + ci * CH
        pltpu.sync_copy(src_hbm.at[pl.ds(off, CH)], iss)
        pltpu.sync_copy(dst_hbm.at[pl.ds(off, CH)], idd)
        pltpu.sync_copy(a_hbm.at[pl.ds(off, CH)], ab)
        pltpu.async_copy(v_hbm.at[iss], vr, gsem)

    def compute(ci, B):
        iss, idd, vrows, abuf, gsem, ssem = B
        pltpu.make_async_copy(v_hbm.at[iss], vrows, gsem).wait()

        def grp(g, gcarry):
            a16 = abuf[pl.ds(g * 16, 16)]

            def edge(l, ecarry):
                jj = g * 16 + l
                idx = lax.broadcast(l, (16,))
                aj = a16.at[idx].get(mode='promise_in_bounds')
                for t in range(D // 16):
                    vrows[jj, pl.ds(16 * t, 16)] = (
                        vrows[jj, pl.ds(16 * t, 16)] * aj)
                return ecarry
            lax.fori_loop(0, 16, edge, 0)
            return gcarry
        lax.fori_loop(0, CH // 16, grp, 0)
        pltpu.async_copy(vrows, vacc.at[idd], ssem, add=True)

    issue(0, bufs[0], False)
    issue(1, bufs[1], False)
    compute(0, bufs[0])
    issue(2, bufs[2], False)
    compute(1, bufs[1])
    issue(3, bufs[0], True)
    compute(2, bufs[2])
    issue(4, bufs[1], True)

    def tri(pi, carry):
        for b in (0, 1, 2):
            ci = 3 * pi + b
            compute(ci, bufs[b])
            issue(ci + 2, bufs[(b + 2) % 3], True)
        return carry
    lax.fori_loop(1, (NCHUNK - 2) // 3, tri, 0)
    compute(NCHUNK - 2, bufs[0])
    compute(NCHUNK - 1, bufs[1])
    for b in (2, 0, 1):
        iss, idd, vr, ab, gsem, ssem = bufs[b]
        pltpu.make_async_copy(vr, vacc.at[idd], ssem).wait()
    plsc.subcore_barrier()
    pltpu.sync_copy(vacc.at[pl.ds(row0, NROW)],
                    vpart_hbm.at[c, pl.ds(row0, NROW)])


def kernel(x, edge_index, edge_attr, params):
    src = edge_index[0].astype(i32)
    dst = edge_index[1].astype(i32)
    e = _edge_embed(edge_attr, params['W_emb'], params['b_emb'])
    h = x
    n_layers = len(params['layers'])
    for li, p in enumerate(params['layers']):
        q, qe, k, v, skip = _proj(h, p)
        alpha, mpart = _sc_alpha(q, qe, k, e, src, dst)
        ex, dpart = _sc_exden(alpha, dst, mpart)
        a, agpart = _sc_anorm(ex, dpart, e, dst)
        vpart = _sc_agg(a, v, src, dst)
        h = _combine(vpart, agpart, skip, p['We'], li < n_layers - 1)
    return h


# speedup vs baseline: 7.6372x; 1.0175x over previous
import functools

import jax
import jax.numpy as jnp
import numpy as np
from jax import lax
from jax.experimental import pallas as pl
from jax.experimental.pallas import tpu as pltpu
from jax.experimental.pallas import tpu_sc as plsc

N = 10000
E = 320000
D = 128
DE = 16
EH = 16

NC = 2
NS = 16
NW = NC * NS
EPW = E // NW
CH = 80
NCHUNK = EPW // CH
NPAD = 10240
SL = NPAD // NS
NROW = N // NS
SCALE = 1.0 / float(np.sqrt(D))
NEG = float("-inf")

_mesh = plsc.VectorSubcoreMesh(
    core_axis_name="c", subcore_axis_name="s", num_cores=NC, num_subcores=NS)

f32 = jnp.float32
i32 = jnp.int32


def _edge_embed(edge_attr, W, b):
    BE = 2000

    def body(ea_ref, w_ref, b_ref, o_ref):
        o_ref[...] = jnp.dot(ea_ref[...], w_ref[...],
                             preferred_element_type=f32) + b_ref[...]

    return pl.pallas_call(
        body,
        grid=(E // BE,),
        in_specs=[pl.BlockSpec((BE, DE), lambda i: (i, 0)),
                  pl.BlockSpec((DE, EH), lambda i: (0, 0)),
                  pl.BlockSpec((1, EH), lambda i: (0, 0))],
        out_specs=pl.BlockSpec((BE, EH), lambda i: (i, 0)),
        out_shape=jax.ShapeDtypeStruct((E, EH), f32),
    )(edge_attr, W, b.reshape(1, EH))


def _proj(h, p):
    BN = 400

    def body(h_ref, wq, bq, wk, bk, wv, bv, ws, bs, we,
             q_ref, qe_ref, k_ref, v_ref, s_ref):
        hh = h_ref[...]
        q = jnp.dot(hh, wq[...], preferred_element_type=f32) + bq[...]
        q_ref[...] = q
        qe_ref[...] = lax.dot_general(q, we[...], (((1,), (1,)), ((), ())),
                                      preferred_element_type=f32)
        k_ref[...] = jnp.dot(hh, wk[...], preferred_element_type=f32) + bk[...]
        v_ref[...] = jnp.dot(hh, wv[...], preferred_element_type=f32) + bv[...]
        s_ref[...] = jnp.dot(hh, ws[...], preferred_element_type=f32) + bs[...]

    row = lambda i: (i, 0)
    fix = lambda i: (0, 0)
    return pl.pallas_call(
        body,
        grid=(N // BN,),
        in_specs=[pl.BlockSpec((BN, D), row),
                  pl.BlockSpec((D, D), fix), pl.BlockSpec((1, D), fix),
                  pl.BlockSpec((D, D), fix), pl.BlockSpec((1, D), fix),
                  pl.BlockSpec((D, D), fix), pl.BlockSpec((1, D), fix),
                  pl.BlockSpec((D, D), fix), pl.BlockSpec((1, D), fix),
                  pl.BlockSpec((EH, D), fix)],
        out_specs=[pl.BlockSpec((BN, D), row), pl.BlockSpec((BN, EH), row),
                   pl.BlockSpec((BN, D), row), pl.BlockSpec((BN, D), row),
                   pl.BlockSpec((BN, D), row)],
        out_shape=[jax.ShapeDtypeStruct((N, D), f32),
                   jax.ShapeDtypeStruct((N, EH), f32),
                   jax.ShapeDtypeStruct((N, D), f32),
                   jax.ShapeDtypeStruct((N, D), f32),
                   jax.ShapeDtypeStruct((N, D), f32)],
    )(h, p['Wq'], p['bq'].reshape(1, D), p['Wk'], p['bk'].reshape(1, D),
      p['Wv'], p['bv'].reshape(1, D), p['Ws'], p['bs'].reshape(1, D), p['We'])


def _combine(vp, ag, skip, we, relu):
    BN = 400

    def body(vp_ref, ag_ref, s_ref, we_ref, o_ref):
        h = vp_ref[0] + vp_ref[1] + s_ref[...]
        agg = ag_ref[0] + ag_ref[1]
        h = h + jnp.dot(agg, we_ref[...], preferred_element_type=f32)
        if relu:
            h = jnp.maximum(h, 0.0)
        o_ref[...] = h

    return pl.pallas_call(
        body,
        grid=(N // BN,),
        in_specs=[pl.BlockSpec((2, BN, D), lambda i: (0, i, 0)),
                  pl.BlockSpec((2, BN, EH), lambda i: (0, i, 0)),
                  pl.BlockSpec((BN, D), lambda i: (i, 0)),
                  pl.BlockSpec((EH, D), lambda i: (0, 0))],
        out_specs=pl.BlockSpec((BN, D), lambda i: (i, 0)),
        out_shape=jax.ShapeDtypeStruct((N, D), f32),
    )(vp, ag, skip, we)


@functools.partial(
    pl.kernel,
    out_type=[jax.ShapeDtypeStruct((E,), f32),
              jax.ShapeDtypeStruct((NC, NPAD), f32)],
    mesh=_mesh,
    compiler_params=pltpu.CompilerParams(needs_layout_passes=False, use_tc_tiling_on_sc=False),
    scratch_types=[
        pltpu.VMEM((CH,), i32),
        pltpu.VMEM((CH,), i32),
        pltpu.VMEM((CH, D), f32),
        pltpu.VMEM((CH, D), f32),
        pltpu.VMEM((CH, EH), f32),
        pltpu.VMEM((CH, EH), f32),
        pltpu.VMEM((CH,), i32),
        pltpu.VMEM((CH,), i32),
        pltpu.VMEM((CH, D), f32),
        pltpu.VMEM((CH, D), f32),
        pltpu.VMEM((CH, EH), f32),
        pltpu.VMEM((CH, EH), f32),
        pltpu.VMEM((CH,), i32),
        pltpu.VMEM((CH,), i32),
        pltpu.VMEM((CH, D), f32),
        pltpu.VMEM((CH, D), f32),
        pltpu.VMEM((CH, EH), f32),
        pltpu.VMEM((CH, EH), f32),
        pltpu.VMEM((CH,), f32),
        pltpu.VMEM((CH,), f32),
        pltpu.VMEM((CH,), f32),
        pltpu.VMEM((NPAD,), f32),
        pltpu.VMEM((SL,), f32),
        pltpu.VMEM((SL,), f32),
        pltpu.SemaphoreType.DMA,
        pltpu.SemaphoreType.DMA,
        pltpu.SemaphoreType.DMA,
        pltpu.SemaphoreType.DMA,
        pltpu.SemaphoreType.DMA,
        pltpu.SemaphoreType.DMA,
        pltpu.VMEM_SHARED((NS, NPAD), f32),
    ],
)
def _sc_alpha(q_hbm, qe_hbm, k_hbm, e_hbm, src_hbm, dst_hbm,
              alpha_hbm, mpart_hbm,
              idx_s0, idx_d0, qrows0, krows0, qerows0, erows0,
              idx_s1, idx_d1, qrows1, krows1, qerows1, erows1,
              idx_s2, idx_d2, qrows2, krows2, qerows2, erows2,
              abuf0, abuf1, abuf2, mloc, acc, tmp,
              sem0, sem1, sem2, stsem0, stsem1, stsem2, msh):
    c = lax.axis_index("c")
    s = lax.axis_index("s")
    wid = c * NS + s
    base = wid * EPW
    bufs = ((idx_s0, idx_d0, qrows0, krows0, qerows0, erows0, sem0,
             abuf0, stsem0),
            (idx_s1, idx_d1, qrows1, krows1, qerows1, erows1, sem1,
             abuf1, stsem1),
            (idx_s2, idx_d2, qrows2, krows2, qerows2, erows2, sem2,
             abuf2, stsem2))

    def issue(ci, B):
        iss, idd, qr, kr, qer, er, sem, ab, stsem = B
        off = base + ci * CH
        pltpu.sync_copy(src_hbm.at[pl.ds(off, CH)], iss)
        pltpu.sync_copy(dst_hbm.at[pl.ds(off, CH)], idd)
        pltpu.sync_copy(e_hbm.at[pl.ds(off, CH)], er)
        pltpu.async_copy(k_hbm.at[iss], kr, sem)
        pltpu.async_copy(q_hbm.at[idd], qr, sem)
        pltpu.async_copy(qe_hbm.at[idd], qer, sem)

    issue(0, bufs[0])
    issue(1, bufs[1])

    def mi(i, carry):
        mloc[pl.ds(i * 16, 16)] = jnp.full((16,), NEG, f32)
        return carry
    lax.fori_loop(0, NPAD // 16, mi, 0)

    def drain(B):
        iss, idd, qr, kr, qer, er, sem, ab, stsem = B
        pltpu.make_async_copy(k_hbm.at[iss], kr, sem).wait()
        pltpu.make_async_copy(q_hbm.at[idd], qr, sem).wait()
        pltpu.make_async_copy(qe_hbm.at[idd], qer, sem).wait()

    def wait_store(ci, B):
        iss, idd, qr, kr, qer, er, sem, ab, stsem = B
        pltpu.make_async_copy(
            ab, alpha_hbm.at[pl.ds(base + ci * CH, CH)], stsem).wait()

    def compute(ci, B):
        iss, idd, qrows, krows, qerows, erows, sem, abuf, stsem = B
        off = base + ci * CH
        lane = lax.broadcasted_iota(i32, (16,), 0)

        def grp(g, gcarry):
            def edge(j, accv):
                jj = g * 16 + j
                av = qrows[jj, pl.ds(0, 16)] * krows[jj, pl.ds(0, 16)]
                for t in range(1, D // 16):
                    av = av + (qrows[jj, pl.ds(16 * t, 16)]
                               * krows[jj, pl.ds(16 * t, 16)])
                av = av + qerows[jj, :] * erows[jj, :]
                for sh in (8, 4, 2, 1):
                    perm = jnp.bitwise_xor(lane, sh)
                    av = av + av.at[perm].get(mode='promise_in_bounds')
                return jnp.where(lane == j, av * SCALE, accv)
            a16 = lax.fori_loop(0, 16, edge, jnp.zeros((16,), f32))
            abuf[pl.ds(g * 16, 16)] = a16
            d16 = idd[pl.ds(g * 16, 16)]

            old = plsc.load_gather(mloc, [d16])
            new = jnp.maximum(old, a16)
            plsc.store_scatter(mloc, [d16], new)
            chk = plsc.load_gather(mloc, [d16])
            pend0 = chk < new

            @pl.when(jnp.any(pend0))
            def _cleanup():
                def bd(t, pend):
                    msk = pend > 0
                    o = plsc.load_gather(mloc, [d16])
                    n = jnp.maximum(o, a16)
                    plsc.store_scatter(mloc, [d16], n, mask=msk)
                    k2 = plsc.load_gather(mloc, [d16])
                    return (msk & (k2 < n)).astype(i32)
                lax.fori_loop(0, 15, bd, pend0.astype(i32))
            return gcarry
        lax.fori_loop(0, CH // 16, grp, 0)
        pltpu.async_copy(abuf, alpha_hbm.at[pl.ds(off, CH)], stsem)

    def tri(pi, carry):
        for b in (0, 1, 2):
            ci = 3 * pi + b
            issue(ci + 2, bufs[(b + 2) % 3])
            drain(bufs[b])

            @pl.when(pi > 0)
            def _ws():
                wait_store(ci - 3, bufs[b])
            compute(ci, bufs[b])
        return carry
    lax.fori_loop(0, (NCHUNK - 2) // 3, tri, 0)
    drain(bufs[0])
    wait_store(NCHUNK - 5, bufs[0])
    compute(NCHUNK - 2, bufs[0])
    drain(bufs[1])
    wait_store(NCHUNK - 4, bufs[1])
    compute(NCHUNK - 1, bufs[1])
    wait_store(NCHUNK - 3, bufs[2])
    wait_store(NCHUNK - 2, bufs[0])
    wait_store(NCHUNK - 1, bufs[1])

    pltpu.sync_copy(mloc, msh.at[s])
    plsc.subcore_barrier()
    col = s * SL
    pltpu.sync_copy(msh.at[0, pl.ds(col, SL)], acc)
    for t in range(1, NS):
        pltpu.sync_copy(msh.at[t, pl.ds(col, SL)], tmp)

        def mx(i, carry):
            acc[pl.ds(i * 16, 16)] = jnp.maximum(acc[pl.ds(i * 16, 16)],
                                                 tmp[pl.ds(i * 16, 16)])
            return carry
        lax.fori_loop(0, SL // 16, mx, 0)
    pltpu.sync_copy(acc, mpart_hbm.at[c, pl.ds(col, SL)])


@functools.partial(
    pl.kernel,
    out_type=[jax.ShapeDtypeStruct((E,), f32),
              jax.ShapeDtypeStruct((NC, NPAD), f32)],
    mesh=_mesh,
    compiler_params=pltpu.CompilerParams(needs_layout_passes=False, use_tc_tiling_on_sc=False),
    scratch_types=[
        pltpu.VMEM((NPAD,), f32),
        pltpu.VMEM((NPAD,), f32),
        pltpu.VMEM((CH,), i32),
        pltpu.VMEM((CH,), f32),
        pltpu.VMEM((CH,), f32),
        pltpu.VMEM((CH,), i32),
        pltpu.VMEM((CH,), f32),
        pltpu.VMEM((CH,), f32),
        pltpu.VMEM((CH,), i32),
        pltpu.VMEM((CH,), f32),
        pltpu.VMEM((CH,), f32),
        pltpu.VMEM((SL,), f32),
        pltpu.SemaphoreType.DMA,
        pltpu.SemaphoreType.DMA,
        pltpu.SemaphoreType.DMA,
        pltpu.SemaphoreType.DMA,
        pltpu.SemaphoreType.DMA,
        pltpu.SemaphoreType.DMA,
        pltpu.VMEM_SHARED((NPAD,), f32),
    ],
)
def _sc_exden(alpha_hbm, dst_hbm, mpart_hbm,
              ex_hbm, dpart_hbm,
              b0, b1,
              idx_d0, abuf0, exbuf0,
              idx_d1, abuf1, exbuf1,
              idx_d2, abuf2, exbuf2,
              zb, ssem0, ssem1, ssem2, esem0, esem1, esem2, dsh):
    c = lax.axis_index("c")
    s = lax.axis_index("s")
    wid = c * NS + s
    base = wid * EPW
    bufs = ((idx_d0, abuf0, exbuf0, ssem0, esem0),
            (idx_d1, abuf1, exbuf1, ssem1, esem1),
            (idx_d2, abuf2, exbuf2, ssem2, esem2))

    pltpu.sync_copy(mpart_hbm.at[0], b0)
    pltpu.sync_copy(mpart_hbm.at[1], b1)

    def mcomb(i, carry):
        m = jnp.maximum(b0[pl.ds(i * 16, 16)], b1[pl.ds(i * 16, 16)])
        fin = (m - m) == 0.0
        b0[pl.ds(i * 16, 16)] = jnp.where(fin, m, 0.0)
        return carry
    lax.fori_loop(0, NPAD // 16, mcomb, 0)

    def z(i, carry):
        zb[pl.ds(i * 16, 16)] = jnp.zeros((16,), f32)
        return carry
    lax.fori_loop(0, SL // 16, z, 0)
    col = s * SL
    pltpu.sync_copy(zb, dsh.at[pl.ds(col, SL)])
    plsc.subcore_barrier()

    def work(ci, B, wait_scatter):
        idx_d, abuf, exbuf, ssem, esem = B
        if wait_scatter:
            pltpu.make_async_copy(exbuf, dsh.at[idx_d], ssem).wait()
            pltpu.make_async_copy(
                exbuf, ex_hbm.at[pl.ds(base + (ci - 3) * CH, CH)],
                esem).wait()
        off = base + ci * CH
        pltpu.sync_copy(dst_hbm.at[pl.ds(off, CH)], idx_d)
        pltpu.sync_copy(alpha_hbm.at[pl.ds(off, CH)], abuf)

        def grp(g, gcarry):
            d16 = idx_d[pl.ds(g * 16, 16)]
            m16 = plsc.load_gather(b0, [d16])
            exbuf[pl.ds(g * 16, 16)] = jnp.exp(abuf[pl.ds(g * 16, 16)] - m16)
            return gcarry
        lax.fori_loop(0, CH // 16, grp, 0)
        pltpu.async_copy(exbuf, ex_hbm.at[pl.ds(off, CH)], esem)
        pltpu.async_copy(exbuf, dsh.at[idx_d], ssem, add=True)

    work(0, bufs[0], False)
    work(1, bufs[1], False)
    work(2, bufs[2], False)

    def tri(pi, carry):
        for b in (0, 1, 2):
            work(3 * pi + b, bufs[b], True)
        return carry
    lax.fori_loop(1, NCHUNK // 3, tri, 0)
    work(NCHUNK - 2, bufs[0], True)
    work(NCHUNK - 1, bufs[1], True)
    for (b, lastci) in ((2, NCHUNK - 3), (0, NCHUNK - 2), (1, NCHUNK - 1)):
        idx_d, abuf, exbuf, ssem, esem = bufs[b]
        pltpu.make_async_copy(exbuf, dsh.at[idx_d], ssem).wait()
        pltpu.make_async_copy(
            exbuf, ex_hbm.at[pl.ds(base + lastci * CH, CH)], esem).wait()
    plsc.subcore_barrier()
    pltpu.sync_copy(dsh.at[pl.ds(col, SL)], dpart_hbm.at[c, pl.ds(col, SL)])


@functools.partial(
    pl.kernel,
    out_type=[jax.ShapeDtypeStruct((E,), f32),
              jax.ShapeDtypeStruct((NC, N, EH), f32)],
    mesh=_mesh,
    compiler_params=pltpu.CompilerParams(needs_layout_passes=False, use_tc_tiling_on_sc=False),
    scratch_types=[
        pltpu.VMEM((NPAD,), f32),
        pltpu.VMEM((NPAD,), f32),
        pltpu.VMEM((CH,), i32),
        pltpu.VMEM((CH, EH), f32),
        pltpu.VMEM((CH,), f32),
        pltpu.VMEM((CH,), f32),
        pltpu.VMEM((CH,), i32),
        pltpu.VMEM((CH, EH), f32),
        pltpu.VMEM((CH,), f32),
        pltpu.VMEM((CH,), f32),
        pltpu.VMEM((CH,), i32),
        pltpu.VMEM((CH, EH), f32),
        pltpu.VMEM((CH,), f32),
        pltpu.VMEM((CH,), f32),
        pltpu.SemaphoreType.DMA,
        pltpu.SemaphoreType.DMA,
        pltpu.SemaphoreType.DMA,
        pltpu.SemaphoreType.DMA,
        pltpu.SemaphoreType.DMA,
        pltpu.SemaphoreType.DMA,
        pltpu.SemaphoreType.DMA,
        pltpu.SemaphoreType.DMA,
        pltpu.SemaphoreType.DMA,
        pltpu.VMEM_SHARED((N, EH), f32),
    ],
)
def _sc_anorm(ex_hbm, dpart_hbm, e_hbm, dst_hbm,
              a_hbm, agpart_hbm,
              d0, d1,
              idx_d0, erows0, exbuf0, abuf0,
              idx_d1, erows1, exbuf1, abuf1,
              idx_d2, erows2, exbuf2, abuf2,
              gsem0, gsem1, gsem2, ssem0, ssem1, ssem2,
              asem0, asem1, asem2, agacc):
    c = lax.axis_index("c")
    s = lax.axis_index("s")
    wid = c * NS + s
    base = wid * EPW
    bufs = ((idx_d0, erows0, exbuf0, abuf0, gsem0, ssem0, asem0),
            (idx_d1, erows1, exbuf1, abuf1, gsem1, ssem1, asem1),
            (idx_d2, erows2, exbuf2, abuf2, gsem2, ssem2, asem2))

    pltpu.sync_copy(dpart_hbm.at[0], d0)
    pltpu.sync_copy(dpart_hbm.at[1], d1)

    def dcomb(i, carry):
        d0[pl.ds(i * 16, 16)] = (d0[pl.ds(i * 16, 16)]
                                 + d1[pl.ds(i * 16, 16)] + 1e-16)
        return carry
    lax.fori_loop(0, NPAD // 16, dcomb, 0)

    def zrow(i, carry):
        erows0[i, :] = jnp.zeros((16,), f32)
        return carry
    lax.fori_loop(0, CH, zrow, 0)

    row0 = s * NROW
    for (st, cnt) in ((0, 80), (80, 80), (160, 80), (240, 80),
                      (320, 80), (400, 80), (480, 80), (560, 65)):
        pltpu.sync_copy(erows0.at[pl.ds(0, cnt)], agacc.at[pl.ds(row0 + st, cnt)])
    plsc.subcore_barrier()

    def issue(ci, B, wait_scatter):
        idd, er, exb, ab, gsem, ssem, asem = B
        if wait_scatter:
            pltpu.make_async_copy(er, agacc.at[idd], ssem).wait()
            pltpu.make_async_copy(
                ab, a_hbm.at[pl.ds(base + (ci - 3) * CH, CH)], asem).wait()
        off = base + ci * CH
        pltpu.sync_copy(dst_hbm.at[pl.ds(off, CH)], idd)
        pltpu.sync_copy(ex_hbm.at[pl.ds(off, CH)], exb)
        pltpu.async_copy(e_hbm.at[pl.ds(off, CH)], er, gsem)

    def compute(ci, B):
        idd, erows, exbuf, abuf, gsem, ssem, asem = B
        off = base + ci * CH
        pltpu.make_async_copy(e_hbm.at[pl.ds(off, CH)], erows, gsem).wait()

        def grp(g, gcarry):
            d16 = idd[pl.ds(g * 16, 16)]
            den = plsc.load_gather(d0, [d16])
            a16 = exbuf[pl.ds(g * 16, 16)] / den
            abuf[pl.ds(g * 16, 16)] = a16

            def edge(l, ecarry):
                jj = g * 16 + l
                idx = lax.broadcast(l, (16,))
                aj = a16.at[idx].get(mode='promise_in_bounds')
                erows[jj, :] = erows[jj, :] * aj
                return ecarry
            lax.fori_loop(0, 16, edge, 0)
            return gcarry
        lax.fori_loop(0, CH // 16, grp, 0)

        pltpu.async_copy(abuf, a_hbm.at[pl.ds(off, CH)], asem)
        pltpu.async_copy(erows, agacc.at[idd], ssem, add=True)

    issue(0, bufs[0], False)
    issue(1, bufs[1], False)
    compute(0, bufs[0])
    issue(2, bufs[2], False)
    compute(1, bufs[1])
    issue(3, bufs[0], True)
    compute(2, bufs[2])
    issue(4, bufs[1], True)

    def tri(pi, carry):
        for b in (0, 1, 2):
            ci = 3 * pi + b
            compute(ci, bufs[b])
            issue(ci + 2, bufs[(b + 2) % 3], True)
        return carry
    lax.fori_loop(1, (NCHUNK - 2) // 3, tri, 0)
    compute(NCHUNK - 2, bufs[0])
    compute(NCHUNK - 1, bufs[1])
    for (b, lastci) in ((2, NCHUNK - 3), (0, NCHUNK - 2), (1, NCHUNK - 1)):
        idd, er, exb, ab, gsem, ssem, asem = bufs[b]
        pltpu.make_async_copy(er, agacc.at[idd], ssem).wait()
        pltpu.make_async_copy(
            ab, a_hbm.at[pl.ds(base + lastci * CH, CH)], asem).wait()
    plsc.subcore_barrier()
    pltpu.sync_copy(agacc.at[pl.ds(row0, NROW)],
                    agpart_hbm.at[c, pl.ds(row0, NROW)])


@functools.partial(
    pl.kernel,
    out_type=jax.ShapeDtypeStruct((NC, N, D), f32),
    mesh=_mesh,
    compiler_params=pltpu.CompilerParams(needs_layout_passes=False, use_tc_tiling_on_sc=False),
    scratch_types=[
        pltpu.VMEM((CH,), i32),
        pltpu.VMEM((CH,), i32),
        pltpu.VMEM((CH, D), f32),
        pltpu.VMEM((CH,), f32),
        pltpu.VMEM((CH,), i32),
        pltpu.VMEM((CH,), i32),
        pltpu.VMEM((CH, D), f32),
        pltpu.VMEM((CH,), f32),
        pltpu.VMEM((CH,), i32),
        pltpu.VMEM((CH,), i32),
        pltpu.VMEM((CH, D), f32),
        pltpu.VMEM((CH,), f32),
        pltpu.SemaphoreType.DMA,
        pltpu.SemaphoreType.DMA,
        pltpu.SemaphoreType.DMA,
        pltpu.SemaphoreType.DMA,
        pltpu.SemaphoreType.DMA,
        pltpu.SemaphoreType.DMA,
        pltpu.VMEM_SHARED((N, D), f32),
    ],
)
def _sc_agg(a_hbm, v_hbm, src_hbm, dst_hbm,
            vpart_hbm,
            idx_s0, idx_d0, vrows0, abuf0,
            idx_s1, idx_d1, vrows1, abuf1,
            idx_s2, idx_d2, vrows2, abuf2,
            gsem0, gsem1, gsem2, ssem0, ssem1, ssem2, vacc):
    c = lax.axis_index("c")
    s = lax.axis_index("s")
    wid = c * NS + s
    base = wid * EPW
    bufs = ((idx_s0, idx_d0, vrows0, abuf0, gsem0, ssem0),
            (idx_s1, idx_d1, vrows1, abuf1, gsem1, ssem1),
            (idx_s2, idx_d2, vrows2, abuf2, gsem2, ssem2))

    def zrow(i, carry):
        for t in range(D // 16):
            vrows0[i, pl.ds(16 * t, 16)] = jnp.zeros((16,), f32)
        return carry
    lax.fori_loop(0, CH, zrow, 0)

    row0 = s * NROW
    for (st, cnt) in ((0, 80), (80, 80), (160, 80), (240, 80),
                      (320, 80), (400, 80), (480, 80), (560, 65)):
        pltpu.sync_copy(vrows0.at[pl.ds(0, cnt)], vacc.at[pl.ds(row0 + st, cnt)])
    plsc.subcore_barrier()

    def issue(ci, B, wait_scatter):
        iss, idd, vr, ab, gsem, ssem = B
        if wait_scatter:
            pltpu.make_async_copy(vr, vacc.at[idd], ssem).wait()
        off = base + ci * CH
        pltpu.sync_copy(src_hbm.at[pl.ds(off, CH)], iss)
        pltpu.sync_copy(dst_hbm.at[pl.ds(off, CH)], idd)
        pltpu.sync_copy(a_hbm.at[pl.ds(off, CH)], ab)
        pltpu.async_copy(v_hbm.at[iss], vr, gsem)

    def compute(ci, B):
        iss, idd, vrows, abuf, gsem, ssem = B
        pltpu.make_async_copy(v_hbm.at[iss], vrows, gsem).wait()

        def grp(g, gcarry):
            a16 = abuf[pl.ds(g * 16, 16)]

            def edge(l, ecarry):
                jj = g * 16 + l
                idx = lax.broadcast(l, (16,))
                aj = a16.at[idx].get(mode='promise_in_bounds')
                for t in range(D // 16):
                    vrows[jj, pl.ds(16 * t, 16)] = (
                        vrows[jj, pl.ds(16 * t, 16)] * aj)
                return ecarry
            lax.fori_loop(0, 16, edge, 0)
            return gcarry
        lax.fori_loop(0, CH // 16, grp, 0)
        pltpu.async_copy(vrows, vacc.at[idd], ssem, add=True)

    issue(0, bufs[0], False)
    issue(1, bufs[1], False)
    compute(0, bufs[0])
    issue(2, bufs[2], False)
    compute(1, bufs[1])
    issue(3, bufs[0], True)
    compute(2, bufs[2])
    issue(4, bufs[1], True)

    def tri(pi, carry):
        for b in (0, 1, 2):
            ci = 3 * pi + b
            compute(ci, bufs[b])
            issue(ci + 2, bufs[(b + 2) % 3], True)
        return carry
    lax.fori_loop(1, (NCHUNK - 2) // 3, tri, 0)
    compute(NCHUNK - 2, bufs[0])
    compute(NCHUNK - 1, bufs[1])
    for b in (2, 0, 1):
        iss, idd, vr, ab, gsem, ssem = bufs[b]
        pltpu.make_async_copy(vr, vacc.at[idd], ssem).wait()
    plsc.subcore_barrier()
    pltpu.sync_copy(vacc.at[pl.ds(row0, NROW)],
                    vpart_hbm.at[c, pl.ds(row0, NROW)])


def kernel(x, edge_index, edge_attr, params):
    src = edge_index[0].astype(i32)
    dst = edge_index[1].astype(i32)
    e = _edge_embed(edge_attr, params['W_emb'], params['b_emb'])
    h = x
    n_layers = len(params['layers'])
    for li, p in enumerate(params['layers']):
        q, qe, k, v, skip = _proj(h, p)
        alpha, mpart = _sc_alpha(q, qe, k, e, src, dst)
        ex, dpart = _sc_exden(alpha, dst, mpart)
        a, agpart = _sc_anorm(ex, dpart, e, dst)
        vpart = _sc_agg(a, v, src, dst)
        h = _combine(vpart, agpart, skip, p['We'], li < n_layers - 1)
    return h
